# Initial kernel scaffold; baseline (speedup 1.0000x reference)
import jax
import jax.numpy as jnp
from jax.experimental import pallas as pl


def kernel(match_mask, pts_3d0, pts_3d1, K0, K1, non_epipolar):
    raise NotImplementedError("write your pallas kernel here")


# speedup vs baseline: 1.0404x; 1.0404x over previous
import jax
import jax.numpy as jnp
from jax.experimental import pallas as pl

N, L, S = 8, 1024, 1024
H, W = 32, 32
A = 128


def _dense_body(p_ref, a_ref, o0_ref, o1_ref):
    p = p_ref[0]
    a = a_ref[0]
    for side, o_ref in ((0, o0_ref), (1, o1_ref)):
        diffs = []
        for c in range(3):
            prow = p[4 * side + c : 4 * side + c + 1, :]
            acol = a[:, 4 * side + c : 4 * side + c + 1]
            diffs.append(prow - acol)
        dist = jnp.sqrt(diffs[0] * diffs[0] + diffs[1] * diffs[1]
                        + diffs[2] * diffs[2])
        feats = diffs + [dist]
        for d, f in enumerate(feats):
            norm = jnp.sum(jnp.abs(f), axis=0, keepdims=True)
            o_ref[0, d * A:(d + 1) * A, :] = f / norm


def _dense_call(P, anchors):
    return pl.pallas_call(
        _dense_body,
        grid=(N,),
        in_specs=[
            pl.BlockSpec((1, 8, L), lambda b: (b, 0, 0)),
            pl.BlockSpec((1, A, 8), lambda b: (b, 0, 0)),
        ],
        out_specs=[
            pl.BlockSpec((1, 4 * A, L), lambda b: (b, 0, 0)),
            pl.BlockSpec((1, 4 * A, L), lambda b: (b, 0, 0)),
        ],
        out_shape=[
            jax.ShapeDtypeStruct((N, 4 * A, L), jnp.float32),
            jax.ShapeDtypeStruct((N, 4 * A, L), jnp.float32),
        ],
    )(P, anchors)


def kernel(match_mask, pts_3d0, pts_3d1, K0, K1, non_epipolar):
    flat = match_mask.reshape(N, L * S)
    _, idx = jax.lax.top_k(flat, A)
    ai = idx // S
    aj = idx % S
    bidx = jnp.arange(N)[:, None]
    anc0 = pts_3d0[bidx, ai]
    anc1 = pts_3d1[bidx, aj]
    z1 = jnp.zeros((N, A, 1), jnp.float32)
    anchors = jnp.concatenate([anc0, z1, anc1, z1], axis=-1)
    z2 = jnp.zeros((N, 1, L), jnp.float32)
    P = jnp.concatenate(
        [pts_3d0.transpose(0, 2, 1), z2, pts_3d1.transpose(0, 2, 1), z2],
        axis=1)
    out0, out1 = _dense_call(P, anchors)
    return (out0.reshape(N, 4 * A, H, W), out1.reshape(N, 4 * A, H, W))


# speedup vs baseline: 2.4171x; 2.3233x over previous
import functools

import jax
import jax.numpy as jnp
from jax import lax
from jax.experimental import pallas as pl
from jax.experimental.pallas import tpu as pltpu
from jax.experimental.pallas import tpu_sc as plsc

N, L, S = 8, 1024, 1024
H, W = 32, 32
A = 128
FLAT = L * S
QUART = FLAT // 4
CH = 16384
NCH = QUART // CH
VPC = CH // 16

I32 = jnp.int32


def _lanes():
    return lax.iota(I32, 16)


def _threshold_find(hist, nb, tgt):
    ng = nb // 16
    lanes = _lanes()
    zero = jnp.zeros((), I32)

    def body(k, carry):
        found, bkt, cnt, acc = carry
        g = ng - 1 - k
        tot = hist[pl.ds(g * 16, 16)]
        for ln in range(1, 16):
            tot = tot + hist[pl.ds(ln * nb + g * 16, 16)]
        rev = lax.rev(tot, (0,))
        csum = plsc.cumsum(rev)
        cross = (acc + csum) >= tgt
        has = jnp.sum(cross.astype(I32)) > 0
        kv = plsc.all_reduce_ffs(cross)
        ks = jnp.max(kv)
        csel = jnp.sum(jnp.where(lanes == ks, csum, 0).astype(I32))
        rsel = jnp.sum(jnp.where(lanes == ks, rev, 0).astype(I32))
        hit = jnp.logical_and(found == 0, has)
        bkt = jnp.where(hit, g * 16 + 15 - ks, bkt)
        cnt = jnp.where(hit, acc + csel - rsel, cnt)
        found = jnp.where(has, jnp.ones((), I32), found)
        acc = jnp.where(found > 0, acc, acc + jnp.sum(tot))
        return found, bkt, cnt, acc

    _, bkt, cnt, _ = lax.fori_loop(0, ng, body,
                                   (zero, zero, zero, zero))
    return bkt, cnt


def _zero_vmem(ref, n):
    z = jnp.zeros((16,), I32)

    def zb(i, c):
        ref[pl.ds(i * 16, 16)] = z
        return c

    lax.fori_loop(0, n // 16, zb, 0)


def _scan_chunks(mm, b, base, buf, fn, carry):
    lanes = _lanes()
    for ci in range(NCH):
        pltpu.sync_copy(mm.at[b, pl.ds(base + ci * CH, CH)], buf)
        cbase = base + ci * CH

        def vbody(i, c, _cbase=cbase):
            x = buf[pl.ds(i * 16, 16)]
            bits = lax.bitcast_convert_type(x, I32)
            gidx = _cbase + i * 16 + lanes
            return fn(bits, gidx, c)

        carry = lax.fori_loop(0, VPC, vbody, carry)
    return carry


def _sc_body(mm, pts0, pts1, out, buf, hist, eqi, cand2, allc, rowbuf,
             mrg, pts0v, pts1v, cands_s, rows_s):
    lanes = _lanes()
    ones = jnp.full((16,), 1, I32)
    cc = lax.axis_index("c")
    ss = lax.axis_index("s")
    grp = ss // 4
    mem = ss % 4
    b = cc * 4 + grp
    base = mem * QUART

    pltpu.sync_copy(pts0.at[b], pts0v)
    pltpu.sync_copy(pts1.at[b], pts1v)

    _zero_vmem(hist, 32768)

    def p1(bits, gidx, c):
        d1 = lax.shift_right_logical(bits, 19)
        plsc.addupdate_scatter(hist, [lanes * 2048 + d1], ones)
        return c

    _scan_chunks(mm, b, base, buf, p1, 0)
    b1, cnt1 = _threshold_find(hist, 2048, 128)
    tgt2 = 128 - cnt1

    _zero_vmem(hist, 32768)

    def p2(bits, gidx, c):
        msk = lax.shift_right_logical(bits, 19) == b1
        d2 = jnp.bitwise_and(lax.shift_right_logical(bits, 8), 0x7FF)
        plsc.addupdate_scatter(hist, [lanes * 2048 + d2], ones, mask=msk)
        return c

    _scan_chunks(mm, b, base, buf, p2, 0)
    b2, cnt2 = _threshold_find(hist, 2048, tgt2)
    tgt3 = tgt2 - cnt2
    p20 = b1 * 2048 + b2

    _zero_vmem(hist, 4096)

    def p3(bits, gidx, c):
        msk = lax.shift_right_logical(bits, 8) == p20
        d3 = jnp.bitwise_and(bits, 0xFF)
        plsc.addupdate_scatter(hist, [lanes * 256 + d3], ones, mask=msk)
        return c

    _scan_chunks(mm, b, base, buf, p3, 0)
    b3, _ = _threshold_find(hist, 256, tgt3)
    thr = p20 * 256 + b3

    neg1 = jnp.full((16,), -1, I32)
    for v in range(10):
        cand2[pl.ds(v * 16, 16)] = neg1
        cand2[pl.ds(160 + v * 16, 16)] = 0x7FF00000 + v * 16 + lanes

    def p4(bits, gidx, c):
        gt_off, eq_off = c
        mg = bits > thr
        plsc.store_compressed(cand2.at[pl.ds(gt_off, 16)], bits, mask=mg)
        plsc.store_compressed(cand2.at[pl.ds(160 + gt_off, 16)], gidx,
                              mask=mg)
        gt_off = gt_off + jnp.sum(mg.astype(I32))
        cap = jnp.full((16,), eq_off, I32) < 128
        me = jnp.logical_and(bits == thr, cap)
        plsc.store_compressed(eqi.at[pl.ds(eq_off, 16)], gidx, mask=me)
        eq_off = eq_off + jnp.sum(me.astype(I32))
        return gt_off, eq_off

    zero = jnp.zeros((), I32)
    count_gt, _ = _scan_chunks(mm, b, base, buf, p4, (zero, zero))
    need_eq = 128 - count_gt

    thr_vec = jnp.full((16,), thr, I32)
    for v in range(8):
        @pl.when(v * 16 < need_eq)
        def _(v=v):
            kk = need_eq - v * 16
            msk = lanes < kk
            ev = eqi[pl.ds(v * 16, 16)]
            plsc.store_compressed(cand2.at[pl.ds(count_gt + v * 16, 16)],
                                  thr_vec, mask=msk)
            plsc.store_compressed(
                cand2.at[pl.ds(160 + count_gt + v * 16, 16)], ev, mask=msk)

    zf = jnp.zeros((16,), jnp.float32)

    def zr(i, c):
        rowbuf[pl.ds(i * 16, 16)] = zf
        return c

    lax.fori_loop(0, 80, zr, 0)

    pltpu.sync_copy(cand2, cands_s.at[pl.ds(grp * 1280 + mem * 320, 320)])
    plsc.subcore_barrier()

    pltpu.sync_copy(cands_s.at[pl.ds(grp * 1280, 1280)], allc)

    def rank_body(v, c):
        ob = allc[pl.ds(mem * 320 + v * 16, 16)]
        oi = allc[pl.ds(mem * 320 + 160 + v * 16, 16)]

        def jt_body(jt, r):
            def w_body(w, r2):
                cb = allc[pl.ds(jt * 320 + w * 16, 16)]
                cv = allc[pl.ds(jt * 320 + 160 + w * 16, 16)]
                for k in range(16):
                    perm = jnp.bitwise_and(lanes + k, 15)
                    rb = cb.at[perm].get(mode="promise_in_bounds",
                                         unique_indices=True)
                    ri = cv.at[perm].get(mode="promise_in_bounds",
                                         unique_indices=True)
                    better = jnp.logical_or(
                        rb > ob,
                        jnp.logical_and(rb == ob, ri < oi))
                    r2 = r2 + better.astype(I32)
                return r2

            return lax.fori_loop(0, 10, w_body, r)

        r = lax.fori_loop(0, 4, jt_body, jnp.zeros((16,), I32))

        safe_i = jnp.where(ob >= 0, oi, 0)
        ii = lax.shift_right_logical(safe_i, 10)
        jj = jnp.bitwise_and(safe_i, 1023)
        tgt = jnp.where(r < 128, r,
                        128 + jnp.bitwise_and(v * 16 + lanes, 31))
        t8 = tgt * 8
        for d in range(3):
            dcol = jnp.full((16,), d, I32)
            g0 = plsc.load_gather(pts0v, [ii * 3 + d])
            plsc.store_scatter(rowbuf, [t8 + d], g0)
            g1 = plsc.load_gather(pts1v, [jj * 3 + d])
            plsc.store_scatter(rowbuf, [t8 + 4 + d], g1)
        return c

    lax.fori_loop(0, 10, rank_body, 0)

    slot = (grp * 4 + mem) * 1280
    pltpu.sync_copy(rowbuf, rows_s.at[pl.ds(slot, 1280)])
    plsc.subcore_barrier()

    @pl.when(mem == 0)
    def _():
        for t in range(1, 4):
            pltpu.sync_copy(rows_s.at[pl.ds((grp * 4 + t) * 1280, 1280)],
                            mrg)

            def madd(i, c):
                rowbuf[pl.ds(i * 16, 16)] = (rowbuf[pl.ds(i * 16, 16)]
                                             + mrg[pl.ds(i * 16, 16)])
                return c

            lax.fori_loop(0, 64, madd, 0)
        pltpu.sync_copy(rowbuf.at[pl.ds(0, 1024)], out.at[b])


def _sc_topk_anchors(mm, pts0f, pts1f):
    mesh = plsc.VectorSubcoreMesh(core_axis_name="c", subcore_axis_name="s")
    fn = pl.kernel(
        _sc_body,
        out_type=jax.ShapeDtypeStruct((N, A * 8), jnp.float32),
        mesh=mesh,
        scratch_types=[
            pltpu.VMEM((CH,), jnp.float32),
            pltpu.VMEM((32768,), I32),
            pltpu.VMEM((160,), I32),
            pltpu.VMEM((320,), I32),
            pltpu.VMEM((1280,), I32),
            pltpu.VMEM((1280,), jnp.float32),
            pltpu.VMEM((1280,), jnp.float32),
            pltpu.VMEM((3 * L,), jnp.float32),
            pltpu.VMEM((3 * L,), jnp.float32),
            pltpu.VMEM_SHARED((5120,), I32),
            pltpu.VMEM_SHARED((16 * 1280,), jnp.float32),
        ],
        compiler_params=pltpu.CompilerParams(needs_layout_passes=False),
    )
    return fn(mm, pts0f, pts1f)


def _dense_body(p_ref, a_ref, o0_ref, o1_ref):
    p = p_ref[0]
    a = a_ref[0]
    for side, o_ref in ((0, o0_ref), (1, o1_ref)):
        diffs = []
        for c in range(3):
            prow = p[4 * side + c: 4 * side + c + 1, :]
            acol = a[:, 4 * side + c: 4 * side + c + 1]
            diffs.append(prow - acol)
        dist = jnp.sqrt(diffs[0] * diffs[0] + diffs[1] * diffs[1]
                        + diffs[2] * diffs[2])
        feats = diffs + [dist]
        for d, f in enumerate(feats):
            norm = jnp.sum(jnp.abs(f), axis=0, keepdims=True)
            o_ref[0, d * A:(d + 1) * A, :] = f / norm


def _dense_call(P, anchors):
    return pl.pallas_call(
        _dense_body,
        grid=(N,),
        in_specs=[
            pl.BlockSpec((1, 8, L), lambda b: (b, 0, 0)),
            pl.BlockSpec((1, A, 8), lambda b: (b, 0, 0)),
        ],
        out_specs=[
            pl.BlockSpec((1, 4 * A, L), lambda b: (b, 0, 0)),
            pl.BlockSpec((1, 4 * A, L), lambda b: (b, 0, 0)),
        ],
        out_shape=[
            jax.ShapeDtypeStruct((N, 4 * A, L), jnp.float32),
            jax.ShapeDtypeStruct((N, 4 * A, L), jnp.float32),
        ],
    )(P, anchors)


def kernel(match_mask, pts_3d0, pts_3d1, K0, K1, non_epipolar):
    mm = match_mask.reshape(N, FLAT)
    anchors = _sc_topk_anchors(mm, pts_3d0.reshape(N, 3 * L),
                               pts_3d1.reshape(N, 3 * L))
    anchors = anchors.reshape(N, A, 8)
    z2 = jnp.zeros((N, 1, L), jnp.float32)
    P = jnp.concatenate(
        [pts_3d0.transpose(0, 2, 1), z2, pts_3d1.transpose(0, 2, 1), z2],
        axis=1)
    out0, out1 = _dense_call(P, anchors)
    return (out0.reshape(N, 4 * A, H, W), out1.reshape(N, 4 * A, H, W))


# speedup vs baseline: 2.5465x; 1.0535x over previous
import functools

import jax
import jax.numpy as jnp
from jax import lax
from jax.experimental import pallas as pl
from jax.experimental.pallas import tpu as pltpu
from jax.experimental.pallas import tpu_sc as plsc

N, L, S = 8, 1024, 1024
H, W = 32, 32
A = 128
FLAT = L * S
QUART = FLAT // 4
CH = 16384
NCH = QUART // CH
VPC = CH // 16

I32 = jnp.int32


def _lanes():
    return lax.iota(I32, 16)


def _threshold_find(hist, nb, tgt):
    ng = nb // 16
    lanes = _lanes()
    zero = jnp.zeros((), I32)

    def body(k, carry):
        found, bkt, cnt, acc = carry
        g = ng - 1 - k
        tot = hist[pl.ds(g * 16, 16)]
        for ln in range(1, 16):
            tot = tot + hist[pl.ds(ln * nb + g * 16, 16)]
        rev = lax.rev(tot, (0,))
        csum = plsc.cumsum(rev)
        cross = (acc + csum) >= tgt
        has = jnp.sum(cross.astype(I32)) > 0
        kv = plsc.all_reduce_ffs(cross)
        ks = jnp.max(kv)
        csel = jnp.sum(jnp.where(lanes == ks, csum, 0).astype(I32))
        rsel = jnp.sum(jnp.where(lanes == ks, rev, 0).astype(I32))
        hit = jnp.logical_and(found == 0, has)
        bkt = jnp.where(hit, g * 16 + 15 - ks, bkt)
        cnt = jnp.where(hit, acc + csel - rsel, cnt)
        found = jnp.where(has, jnp.ones((), I32), found)
        acc = jnp.where(found > 0, acc, acc + jnp.sum(tot))
        return found, bkt, cnt, acc

    _, bkt, cnt, _ = lax.fori_loop(0, ng, body,
                                   (zero, zero, zero, zero))
    return bkt, cnt


def _zero_vmem(ref, n):
    z = jnp.zeros((16,), I32)

    def zb(i, c):
        ref[pl.ds(i * 16, 16)] = z
        return c

    lax.fori_loop(0, n // 16, zb, 0)


_UNROLL = 4


def _scan_chunks(mm, b, base, buf, fn, carry):
    lanes = _lanes()
    for ci in range(NCH):
        pltpu.sync_copy(mm.at[b, pl.ds(base + ci * CH, CH)], buf)
        cbase = base + ci * CH

        def vbody(i, c, _cbase=cbase):
            for u in range(_UNROLL):
                off = i * (16 * _UNROLL) + u * 16
                x = buf[pl.ds(off, 16)]
                bits = lax.bitcast_convert_type(x, I32)
                gidx = _cbase + off + lanes
                c = fn(bits, gidx, c)
            return c

        carry = lax.fori_loop(0, VPC // _UNROLL, vbody, carry)
    return carry


def _sc_body(mm, pts0, pts1, out, buf, hist, eqi, cand2, allc, rowbuf,
             mrg, pts0v, pts1v, cands_s, rows_s):
    lanes = _lanes()
    ones = jnp.full((16,), 1, I32)
    cc = lax.axis_index("c")
    ss = lax.axis_index("s")
    grp = ss // 4
    mem = ss % 4
    b = cc * 4 + grp
    base = mem * QUART

    pltpu.sync_copy(pts0.at[b], pts0v)
    pltpu.sync_copy(pts1.at[b], pts1v)

    _zero_vmem(hist, 32768)

    def p1(bits, gidx, c):
        d1 = lax.shift_right_logical(bits, 19)
        plsc.addupdate_scatter(hist, [lanes * 2048 + d1], ones)
        return c

    _scan_chunks(mm, b, base, buf, p1, 0)
    b1, cnt1 = _threshold_find(hist, 2048, 128)
    tgt2 = 128 - cnt1

    _zero_vmem(hist, 32768)

    def p2(bits, gidx, c):
        msk = lax.shift_right_logical(bits, 19) == b1
        d2 = jnp.bitwise_and(lax.shift_right_logical(bits, 8), 0x7FF)
        plsc.addupdate_scatter(hist, [lanes * 2048 + d2], ones, mask=msk)
        return c

    _scan_chunks(mm, b, base, buf, p2, 0)
    b2, cnt2 = _threshold_find(hist, 2048, tgt2)
    tgt3 = tgt2 - cnt2
    p20 = b1 * 2048 + b2

    _zero_vmem(hist, 4096)

    def p3(bits, gidx, c):
        msk = lax.shift_right_logical(bits, 8) == p20
        d3 = jnp.bitwise_and(bits, 0xFF)
        plsc.addupdate_scatter(hist, [lanes * 256 + d3], ones, mask=msk)
        return c

    _scan_chunks(mm, b, base, buf, p3, 0)
    b3, _ = _threshold_find(hist, 256, tgt3)
    thr = p20 * 256 + b3

    neg1 = jnp.full((16,), -1, I32)
    for v in range(10):
        cand2[pl.ds(v * 16, 16)] = neg1
        cand2[pl.ds(160 + v * 16, 16)] = 0x7FF00000 + v * 16 + lanes

    def p4(bits, gidx, c):
        gt_off, eq_off = c
        mg = bits > thr
        plsc.store_compressed(cand2.at[pl.ds(gt_off, 16)], bits, mask=mg)
        plsc.store_compressed(cand2.at[pl.ds(160 + gt_off, 16)], gidx,
                              mask=mg)
        gt_off = gt_off + jnp.sum(mg.astype(I32))
        cap = jnp.full((16,), eq_off, I32) < 128
        me = jnp.logical_and(bits == thr, cap)
        plsc.store_compressed(eqi.at[pl.ds(eq_off, 16)], gidx, mask=me)
        eq_off = eq_off + jnp.sum(me.astype(I32))
        return gt_off, eq_off

    zero = jnp.zeros((), I32)
    count_gt, _ = _scan_chunks(mm, b, base, buf, p4, (zero, zero))
    need_eq = 128 - count_gt

    thr_vec = jnp.full((16,), thr, I32)
    for v in range(8):
        @pl.when(v * 16 < need_eq)
        def _(v=v):
            kk = need_eq - v * 16
            msk = lanes < kk
            ev = eqi[pl.ds(v * 16, 16)]
            plsc.store_compressed(cand2.at[pl.ds(count_gt + v * 16, 16)],
                                  thr_vec, mask=msk)
            plsc.store_compressed(
                cand2.at[pl.ds(160 + count_gt + v * 16, 16)], ev, mask=msk)

    zf = jnp.zeros((16,), jnp.float32)

    def zr(i, c):
        rowbuf[pl.ds(i * 16, 16)] = zf
        return c

    lax.fori_loop(0, 80, zr, 0)

    pltpu.sync_copy(cand2, cands_s.at[pl.ds(grp * 1280 + mem * 320, 320)])
    plsc.subcore_barrier()

    pltpu.sync_copy(cands_s.at[pl.ds(grp * 1280, 1280)], allc)

    def rank_body(v, c):
        ob = allc[pl.ds(mem * 320 + v * 16, 16)]
        oi = allc[pl.ds(mem * 320 + 160 + v * 16, 16)]

        def jt_body(jt, r):
            def w_body(w, r2):
                cb = allc[pl.ds(jt * 320 + w * 16, 16)]
                cv = allc[pl.ds(jt * 320 + 160 + w * 16, 16)]
                for k in range(16):
                    perm = jnp.bitwise_and(lanes + k, 15)
                    rb = cb.at[perm].get(mode="promise_in_bounds",
                                         unique_indices=True)
                    ri = cv.at[perm].get(mode="promise_in_bounds",
                                         unique_indices=True)
                    better = jnp.logical_or(
                        rb > ob,
                        jnp.logical_and(rb == ob, ri < oi))
                    r2 = r2 + better.astype(I32)
                return r2

            return lax.fori_loop(0, 10, w_body, r)

        r = lax.fori_loop(0, 4, jt_body, jnp.zeros((16,), I32))

        safe_i = jnp.where(ob >= 0, oi, 0)
        ii = lax.shift_right_logical(safe_i, 10)
        jj = jnp.bitwise_and(safe_i, 1023)
        tgt = jnp.where(r < 128, r,
                        128 + jnp.bitwise_and(v * 16 + lanes, 31))
        t8 = tgt * 8
        for d in range(3):
            dcol = jnp.full((16,), d, I32)
            g0 = plsc.load_gather(pts0v, [ii * 3 + d])
            plsc.store_scatter(rowbuf, [t8 + d], g0)
            g1 = plsc.load_gather(pts1v, [jj * 3 + d])
            plsc.store_scatter(rowbuf, [t8 + 4 + d], g1)
        return c

    lax.fori_loop(0, 10, rank_body, 0)

    slot = (grp * 4 + mem) * 1280
    pltpu.sync_copy(rowbuf, rows_s.at[pl.ds(slot, 1280)])
    plsc.subcore_barrier()

    @pl.when(mem == 0)
    def _():
        for t in range(1, 4):
            pltpu.sync_copy(rows_s.at[pl.ds((grp * 4 + t) * 1280, 1280)],
                            mrg)

            def madd(i, c):
                rowbuf[pl.ds(i * 16, 16)] = (rowbuf[pl.ds(i * 16, 16)]
                                             + mrg[pl.ds(i * 16, 16)])
                return c

            lax.fori_loop(0, 64, madd, 0)
        pltpu.sync_copy(rowbuf.at[pl.ds(0, 1024)], out.at[b])


def _sc_topk_anchors(mm, pts0f, pts1f):
    mesh = plsc.VectorSubcoreMesh(core_axis_name="c", subcore_axis_name="s")
    fn = pl.kernel(
        _sc_body,
        out_type=jax.ShapeDtypeStruct((N, A * 8), jnp.float32),
        mesh=mesh,
        scratch_types=[
            pltpu.VMEM((CH,), jnp.float32),
            pltpu.VMEM((32768,), I32),
            pltpu.VMEM((160,), I32),
            pltpu.VMEM((320,), I32),
            pltpu.VMEM((1280,), I32),
            pltpu.VMEM((1280,), jnp.float32),
            pltpu.VMEM((1280,), jnp.float32),
            pltpu.VMEM((3 * L,), jnp.float32),
            pltpu.VMEM((3 * L,), jnp.float32),
            pltpu.VMEM_SHARED((5120,), I32),
            pltpu.VMEM_SHARED((16 * 1280,), jnp.float32),
        ],
        compiler_params=pltpu.CompilerParams(needs_layout_passes=False),
    )
    return fn(mm, pts0f, pts1f)


def _dense_body(p_ref, a_ref, o0_ref, o1_ref):
    p = p_ref[0]
    a = a_ref[0]
    for side, o_ref in ((0, o0_ref), (1, o1_ref)):
        diffs = []
        for c in range(3):
            prow = p[4 * side + c: 4 * side + c + 1, :]
            acol = a[:, 4 * side + c: 4 * side + c + 1]
            diffs.append(prow - acol)
        dist = jnp.sqrt(diffs[0] * diffs[0] + diffs[1] * diffs[1]
                        + diffs[2] * diffs[2])
        feats = diffs + [dist]
        for d, f in enumerate(feats):
            norm = jnp.sum(jnp.abs(f), axis=0, keepdims=True)
            o_ref[0, d * A:(d + 1) * A, :] = f / norm


def _dense_call(P, anchors):
    return pl.pallas_call(
        _dense_body,
        grid=(N,),
        in_specs=[
            pl.BlockSpec((1, 8, L), lambda b: (b, 0, 0)),
            pl.BlockSpec((1, A, 8), lambda b: (b, 0, 0)),
        ],
        out_specs=[
            pl.BlockSpec((1, 4 * A, L), lambda b: (b, 0, 0)),
            pl.BlockSpec((1, 4 * A, L), lambda b: (b, 0, 0)),
        ],
        out_shape=[
            jax.ShapeDtypeStruct((N, 4 * A, L), jnp.float32),
            jax.ShapeDtypeStruct((N, 4 * A, L), jnp.float32),
        ],
    )(P, anchors)


def kernel(match_mask, pts_3d0, pts_3d1, K0, K1, non_epipolar):
    mm = match_mask.reshape(N, FLAT)
    anchors = _sc_topk_anchors(mm, pts_3d0.reshape(N, 3 * L),
                               pts_3d1.reshape(N, 3 * L))
    anchors = anchors.reshape(N, A, 8)
    z2 = jnp.zeros((N, 1, L), jnp.float32)
    P = jnp.concatenate(
        [pts_3d0.transpose(0, 2, 1), z2, pts_3d1.transpose(0, 2, 1), z2],
        axis=1)
    out0, out1 = _dense_call(P, anchors)
    return (out0.reshape(N, 4 * A, H, W), out1.reshape(N, 4 * A, H, W))


# speedup vs baseline: 3.7347x; 1.4666x over previous
import functools

import jax
import jax.numpy as jnp
from jax import lax
from jax.experimental import pallas as pl
from jax.experimental.pallas import tpu as pltpu
from jax.experimental.pallas import tpu_sc as plsc

N, L, S = 8, 1024, 1024
H, W = 32, 32
A = 128
FLAT = L * S
QUART = FLAT // 4
CH = 16384
NCH = QUART // CH
VPC = CH // 16

I32 = jnp.int32


def _lanes():
    return lax.iota(I32, 16)


def _threshold_find(hist, nb, tgt):
    ng = nb // 16
    lanes = _lanes()
    zero = jnp.zeros((), I32)

    def body(k, carry):
        found, bkt, cnt, acc = carry
        g = ng - 1 - k
        tot = hist[pl.ds(g * 16, 16)]
        for ln in range(1, 16):
            tot = tot + hist[pl.ds(ln * nb + g * 16, 16)]
        rev = lax.rev(tot, (0,))
        csum = plsc.cumsum(rev)
        cross = (acc + csum) >= tgt
        has = jnp.sum(cross.astype(I32)) > 0
        kv = plsc.all_reduce_ffs(cross)
        ks = jnp.max(kv)
        csel = jnp.sum(jnp.where(lanes == ks, csum, 0).astype(I32))
        rsel = jnp.sum(jnp.where(lanes == ks, rev, 0).astype(I32))
        hit = jnp.logical_and(found == 0, has)
        bkt = jnp.where(hit, g * 16 + 15 - ks, bkt)
        cnt = jnp.where(hit, acc + csel - rsel, cnt)
        found = jnp.where(has, jnp.ones((), I32), found)
        acc = jnp.where(found > 0, acc, acc + jnp.sum(tot))
        return found, bkt, cnt, acc

    _, bkt, cnt, _ = lax.fori_loop(0, ng, body,
                                   (zero, zero, zero, zero))
    return bkt, cnt


def _zero_vmem(ref, n):
    z = jnp.zeros((16,), I32)

    def zb(i, c):
        ref[pl.ds(i * 16, 16)] = z
        return c

    lax.fori_loop(0, n // 16, zb, 0)


_UNROLL = 4


def _scan_chunks(mm, b, base, buf, fn, carry, unroll=_UNROLL):
    lanes = _lanes()
    for ci in range(NCH):
        pltpu.sync_copy(mm.at[b, pl.ds(base + ci * CH, CH)], buf)
        cbase = base + ci * CH

        def vbody(i, c, _cbase=cbase):
            for u in range(unroll):
                off = i * (16 * unroll) + u * 16
                x = buf[pl.ds(off, 16)]
                bits = lax.bitcast_convert_type(x, I32)
                gidx = _cbase + off + lanes
                c = fn(bits, gidx, c)
            return c

        carry = lax.fori_loop(0, VPC // unroll, vbody, carry)
    return carry


def _scan_coll(colb, coli, n, fn, carry, unroll=_UNROLL):
    lanes = _lanes()

    def vbody(i, c):
        for u in range(unroll):
            off = i * (16 * unroll) + u * 16
            x = colb[pl.ds(off, 16)]
            gidx = coli[pl.ds(off, 16)]
            valid = (off + lanes) < n
            c = fn(x, gidx, valid, c)
        return c

    nv = (n + 16 * unroll - 1) // (16 * unroll)
    return lax.fori_loop(0, nv, vbody, carry)


_CAP = 32752


def _sc_body(mm, pts0, pts1, out, buf, hist, colb, coli, eqi, cand2, allc,
             rowbuf, mrg, pts0v, pts1v, smem, cands_s, rows_s):
    lanes = _lanes()
    ones = jnp.full((16,), 1, I32)
    zero = jnp.zeros((), I32)
    cc = lax.axis_index("c")
    ss = lax.axis_index("s")
    grp = ss // 4
    mem = ss % 4
    b = cc * 4 + grp
    base = mem * QUART

    pltpu.sync_copy(pts0.at[b], pts0v)
    pltpu.sync_copy(pts1.at[b], pts1v)

    _zero_vmem(hist, 32768)

    def p1(bits, gidx, c):
        d1 = lax.shift_right_logical(bits, 19)
        plsc.addupdate_scatter(hist, [lanes * 2048 + d1], ones)
        return c

    _scan_chunks(mm, b, base, buf, p1, 0)
    b1, cnt1 = _threshold_find(hist, 2048, 128)
    tgt2 = 128 - cnt1

    def p2c(bits, gidx, c):
        n_st, n_ge = c
        mge = lax.shift_right_logical(bits, 19) >= b1
        mst = jnp.logical_and(mge, jnp.full((16,), n_st, I32) < _CAP)
        plsc.store_compressed(colb.at[pl.ds(n_st, 16)], bits, mask=mst)
        plsc.store_compressed(coli.at[pl.ds(n_st, 16)], gidx, mask=mst)
        n_st = n_st + jnp.sum(mst.astype(I32))
        n_ge = n_ge + jnp.sum(mge.astype(I32))
        return n_st, n_ge

    n_st, n_ge = _scan_chunks(mm, b, base, buf, p2c, (zero, zero))
    ok = n_st == n_ge

    _zero_vmem(hist, 32768)

    @pl.when(ok)
    def _():
        def f2(x, gidx, valid, c):
            msk = jnp.logical_and(
                valid, lax.shift_right_logical(x, 19) == b1)
            d2 = jnp.bitwise_and(lax.shift_right_logical(x, 8), 0x7FF)
            plsc.addupdate_scatter(hist, [lanes * 2048 + d2], ones,
                                   mask=msk)
            return c

        _scan_coll(colb, coli, n_st, f2, 0)

    @pl.when(jnp.logical_not(ok))
    def _():
        def p2(bits, gidx, c):
            msk = lax.shift_right_logical(bits, 19) == b1
            d2 = jnp.bitwise_and(lax.shift_right_logical(bits, 8), 0x7FF)
            plsc.addupdate_scatter(hist, [lanes * 2048 + d2], ones,
                                   mask=msk)
            return c

        _scan_chunks(mm, b, base, buf, p2, 0, unroll=1)

    b2, cnt2 = _threshold_find(hist, 2048, tgt2)
    tgt3 = tgt2 - cnt2
    p20 = b1 * 2048 + b2

    _zero_vmem(hist, 4096)

    @pl.when(ok)
    def _():
        def f3(x, gidx, valid, c):
            msk = jnp.logical_and(
                valid, lax.shift_right_logical(x, 8) == p20)
            d3 = jnp.bitwise_and(x, 0xFF)
            plsc.addupdate_scatter(hist, [lanes * 256 + d3], ones,
                                   mask=msk)
            return c

        _scan_coll(colb, coli, n_st, f3, 0)

    @pl.when(jnp.logical_not(ok))
    def _():
        def p3(bits, gidx, c):
            msk = lax.shift_right_logical(bits, 8) == p20
            d3 = jnp.bitwise_and(bits, 0xFF)
            plsc.addupdate_scatter(hist, [lanes * 256 + d3], ones,
                                   mask=msk)
            return c

        _scan_chunks(mm, b, base, buf, p3, 0, unroll=1)

    b3, _ = _threshold_find(hist, 256, tgt3)
    thr = p20 * 256 + b3

    neg1 = jnp.full((16,), -1, I32)
    for v in range(10):
        cand2[pl.ds(v * 16, 16)] = neg1
        cand2[pl.ds(160 + v * 16, 16)] = 0x7FF00000 + v * 16 + lanes

    def p4(bits, gidx, valid, c):
        gt_off, eq_off = c
        mg = jnp.logical_and(valid, bits > thr)
        plsc.store_compressed(cand2.at[pl.ds(gt_off, 16)], bits, mask=mg)
        plsc.store_compressed(cand2.at[pl.ds(160 + gt_off, 16)], gidx,
                              mask=mg)
        gt_off = gt_off + jnp.sum(mg.astype(I32))
        cap = jnp.full((16,), eq_off, I32) < 128
        me = jnp.logical_and(jnp.logical_and(valid, bits == thr), cap)
        plsc.store_compressed(eqi.at[pl.ds(eq_off, 16)], gidx, mask=me)
        eq_off = eq_off + jnp.sum(me.astype(I32))
        return gt_off, eq_off

    @pl.when(ok)
    def _():
        gt_off, _eq = _scan_coll(colb, coli, n_st, p4, (zero, zero))
        smem[0] = gt_off

    @pl.when(jnp.logical_not(ok))
    def _():
        def p4f(bits, gidx, c):
            tv = jnp.full((16,), 1, I32) > 0
            return p4(bits, gidx, tv, c)

        gt_off, _eq = _scan_chunks(mm, b, base, buf, p4f, (zero, zero),
                                   unroll=1)
        smem[0] = gt_off

    count_gt = smem[0]
    need_eq = 128 - count_gt

    thr_vec = jnp.full((16,), thr, I32)
    for v in range(8):
        @pl.when(v * 16 < need_eq)
        def _(v=v):
            kk = need_eq - v * 16
            msk = lanes < kk
            ev = eqi[pl.ds(v * 16, 16)]
            plsc.store_compressed(cand2.at[pl.ds(count_gt + v * 16, 16)],
                                  thr_vec, mask=msk)
            plsc.store_compressed(
                cand2.at[pl.ds(160 + count_gt + v * 16, 16)], ev, mask=msk)

    zf = jnp.zeros((16,), jnp.float32)

    def zr(i, c):
        rowbuf[pl.ds(i * 16, 16)] = zf
        return c

    lax.fori_loop(0, 80, zr, 0)

    pltpu.sync_copy(cand2, cands_s.at[pl.ds(grp * 1280 + mem * 320, 320)])
    plsc.subcore_barrier()

    pltpu.sync_copy(cands_s.at[pl.ds(grp * 1280, 1280)], allc)

    def rank_body(v, c):
        ob = allc[pl.ds(mem * 320 + v * 16, 16)]
        oi = allc[pl.ds(mem * 320 + 160 + v * 16, 16)]

        def jt_body(jt, r):
            def w_body(w, r2):
                cb = allc[pl.ds(jt * 320 + w * 16, 16)]
                cv = allc[pl.ds(jt * 320 + 160 + w * 16, 16)]
                for k in range(16):
                    perm = jnp.bitwise_and(lanes + k, 15)
                    rb = cb.at[perm].get(mode="promise_in_bounds",
                                         unique_indices=True)
                    ri = cv.at[perm].get(mode="promise_in_bounds",
                                         unique_indices=True)
                    better = jnp.logical_or(
                        rb > ob,
                        jnp.logical_and(rb == ob, ri < oi))
                    r2 = r2 + better.astype(I32)
                return r2

            return lax.fori_loop(0, 10, w_body, r)

        r = lax.fori_loop(0, 4, jt_body, jnp.zeros((16,), I32))

        safe_i = jnp.where(ob >= 0, oi, 0)
        ii = lax.shift_right_logical(safe_i, 10)
        jj = jnp.bitwise_and(safe_i, 1023)
        tgt = jnp.where(r < 128, r,
                        128 + jnp.bitwise_and(v * 16 + lanes, 31))
        t8 = tgt * 8
        for d in range(3):
            dcol = jnp.full((16,), d, I32)
            g0 = plsc.load_gather(pts0v, [ii * 3 + d])
            plsc.store_scatter(rowbuf, [t8 + d], g0)
            g1 = plsc.load_gather(pts1v, [jj * 3 + d])
            plsc.store_scatter(rowbuf, [t8 + 4 + d], g1)
        return c

    lax.fori_loop(0, 10, rank_body, 0)

    slot = (grp * 4 + mem) * 1280
    pltpu.sync_copy(rowbuf, rows_s.at[pl.ds(slot, 1280)])
    plsc.subcore_barrier()

    @pl.when(mem == 0)
    def _():
        for t in range(1, 4):
            pltpu.sync_copy(rows_s.at[pl.ds((grp * 4 + t) * 1280, 1280)],
                            mrg)

            def madd(i, c):
                rowbuf[pl.ds(i * 16, 16)] = (rowbuf[pl.ds(i * 16, 16)]
                                             + mrg[pl.ds(i * 16, 16)])
                return c

            lax.fori_loop(0, 64, madd, 0)
        pltpu.sync_copy(rowbuf.at[pl.ds(0, 1024)], out.at[b])


def _sc_topk_anchors(mm, pts0f, pts1f):
    mesh = plsc.VectorSubcoreMesh(core_axis_name="c", subcore_axis_name="s")
    fn = pl.kernel(
        _sc_body,
        out_type=jax.ShapeDtypeStruct((N, A * 8), jnp.float32),
        mesh=mesh,
        scratch_types=[
            pltpu.VMEM((CH,), jnp.float32),
            pltpu.VMEM((32768,), I32),
            pltpu.VMEM((32832,), I32),
            pltpu.VMEM((32832,), I32),
            pltpu.VMEM((160,), I32),
            pltpu.VMEM((320,), I32),
            pltpu.VMEM((1280,), I32),
            pltpu.VMEM((1280,), jnp.float32),
            pltpu.VMEM((1280,), jnp.float32),
            pltpu.VMEM((3 * L,), jnp.float32),
            pltpu.VMEM((3 * L,), jnp.float32),
            pltpu.SMEM((8,), I32),
            pltpu.VMEM_SHARED((5120,), I32),
            pltpu.VMEM_SHARED((16 * 1280,), jnp.float32),
        ],
        compiler_params=pltpu.CompilerParams(needs_layout_passes=False),
    )
    return fn(mm, pts0f, pts1f)


def _dense_body(p_ref, a_ref, o0_ref, o1_ref):
    p = p_ref[0]
    a = a_ref[0]
    for side, o_ref in ((0, o0_ref), (1, o1_ref)):
        diffs = []
        for c in range(3):
            prow = p[4 * side + c: 4 * side + c + 1, :]
            acol = a[:, 4 * side + c: 4 * side + c + 1]
            diffs.append(prow - acol)
        dist = jnp.sqrt(diffs[0] * diffs[0] + diffs[1] * diffs[1]
                        + diffs[2] * diffs[2])
        feats = diffs + [dist]
        for d, f in enumerate(feats):
            norm = jnp.sum(jnp.abs(f), axis=0, keepdims=True)
            o_ref[0, d * A:(d + 1) * A, :] = f / norm


def _dense_call(P, anchors):
    return pl.pallas_call(
        _dense_body,
        grid=(N,),
        in_specs=[
            pl.BlockSpec((1, 8, L), lambda b: (b, 0, 0)),
            pl.BlockSpec((1, A, 8), lambda b: (b, 0, 0)),
        ],
        out_specs=[
            pl.BlockSpec((1, 4 * A, L), lambda b: (b, 0, 0)),
            pl.BlockSpec((1, 4 * A, L), lambda b: (b, 0, 0)),
        ],
        out_shape=[
            jax.ShapeDtypeStruct((N, 4 * A, L), jnp.float32),
            jax.ShapeDtypeStruct((N, 4 * A, L), jnp.float32),
        ],
    )(P, anchors)


def kernel(match_mask, pts_3d0, pts_3d1, K0, K1, non_epipolar):
    mm = match_mask.reshape(N, FLAT)
    anchors = _sc_topk_anchors(mm, pts_3d0.reshape(N, 3 * L),
                               pts_3d1.reshape(N, 3 * L))
    anchors = anchors.reshape(N, A, 8)
    z2 = jnp.zeros((N, 1, L), jnp.float32)
    P = jnp.concatenate(
        [pts_3d0.transpose(0, 2, 1), z2, pts_3d1.transpose(0, 2, 1), z2],
        axis=1)
    out0, out1 = _dense_call(P, anchors)
    return (out0.reshape(N, 4 * A, H, W), out1.reshape(N, 4 * A, H, W))


# speedup vs baseline: 4.8652x; 1.3027x over previous
import functools

import jax
import jax.numpy as jnp
from jax import lax
from jax.experimental import pallas as pl
from jax.experimental.pallas import tpu as pltpu
from jax.experimental.pallas import tpu_sc as plsc

N, L, S = 8, 1024, 1024
H, W = 32, 32
A = 128
FLAT = L * S
QUART = FLAT // 4
CH = 16384
NCH = QUART // CH
VPC = CH // 16

I32 = jnp.int32


def _lanes():
    return lax.iota(I32, 16)


def _threshold_find(hist, nb, tgt):
    ng = nb // 16
    lanes = _lanes()
    zero = jnp.zeros((), I32)

    def body(k, carry):
        found, bkt, cnt, acc = carry
        g = ng - 1 - k
        tot = hist[pl.ds(g * 16, 16)]
        for ln in range(1, 16):
            tot = tot + hist[pl.ds(ln * nb + g * 16, 16)]
        rev = lax.rev(tot, (0,))
        csum = plsc.cumsum(rev)
        cross = (acc + csum) >= tgt
        has = jnp.sum(cross.astype(I32)) > 0
        kv = plsc.all_reduce_ffs(cross)
        ks = jnp.max(kv)
        csel = jnp.sum(jnp.where(lanes == ks, csum, 0).astype(I32))
        rsel = jnp.sum(jnp.where(lanes == ks, rev, 0).astype(I32))
        hit = jnp.logical_and(found == 0, has)
        bkt = jnp.where(hit, g * 16 + 15 - ks, bkt)
        cnt = jnp.where(hit, acc + csel - rsel, cnt)
        found = jnp.where(has, jnp.ones((), I32), found)
        acc = jnp.where(found > 0, acc, acc + jnp.sum(tot))
        return found, bkt, cnt, acc

    _, bkt, cnt, _ = lax.fori_loop(0, ng, body,
                                   (zero, zero, zero, zero))
    return bkt, cnt


def _zero_vmem(ref, n):
    z = jnp.zeros((16,), I32)

    def zb(i, c):
        ref[pl.ds(i * 16, 16)] = z
        return c

    lax.fori_loop(0, n // 16, zb, 0)


_UNROLL = 4


def _scan_chunks(mm, b, base, buf, fn, carry, unroll=_UNROLL):
    lanes = _lanes()
    for ci in range(NCH):
        pltpu.sync_copy(mm.at[b, pl.ds(base + ci * CH, CH)], buf)
        cbase = base + ci * CH

        def vbody(i, c, _cbase=cbase):
            for u in range(unroll):
                off = i * (16 * unroll) + u * 16
                x = buf[pl.ds(off, 16)]
                bits = lax.bitcast_convert_type(x, I32)
                gidx = _cbase + off + lanes
                c = fn(bits, gidx, c)
            return c

        carry = lax.fori_loop(0, VPC // unroll, vbody, carry)
    return carry


def _scan_coll(colb, coli, n, fn, carry, unroll=_UNROLL):
    lanes = _lanes()

    def vbody(i, c):
        for u in range(unroll):
            off = i * (16 * unroll) + u * 16
            x = colb[pl.ds(off, 16)]
            gidx = coli[pl.ds(off, 16)]
            valid = (off + lanes) < n
            c = fn(x, gidx, valid, c)
        return c

    nv = (n + 16 * unroll - 1) // (16 * unroll)
    return lax.fori_loop(0, nv, vbody, carry)


_CAP = 32752


def _sc_body(mm, pts0, pts1, out, buf, hist, colb, coli, eqi, cand2, allc,
             rowbuf, mrg, pts0v, pts1v, smem, cands_s, rows_s):
    lanes = _lanes()
    ones = jnp.full((16,), 1, I32)
    zero = jnp.zeros((), I32)
    cc = lax.axis_index("c")
    ss = lax.axis_index("s")
    grp = ss // 4
    mem = ss % 4
    b = cc * 4 + grp
    base = mem * QUART

    pltpu.sync_copy(pts0.at[b], pts0v)
    pltpu.sync_copy(pts1.at[b], pts1v)

    with jax.named_scope("p1_fused"):
        _zero_vmem(hist, 32768)

        def compact_fn(bits, gidx, tb, c):
            n_st, n_ge = c
            mge = lax.shift_right_logical(bits, 19) >= tb
            mst = jnp.logical_and(mge, jnp.full((16,), n_st, I32) < _CAP)
            plsc.store_compressed(colb.at[pl.ds(n_st, 16)], bits, mask=mst)
            plsc.store_compressed(coli.at[pl.ds(n_st, 16)], gidx, mask=mst)
            n_st = n_st + jnp.sum(mst.astype(I32))
            n_ge = n_ge + jnp.sum(mge.astype(I32))
            return n_st, n_ge

        pltpu.sync_copy(mm.at[b, pl.ds(base, CH)], buf)

        def ph(i, c):
            for u in range(_UNROLL):
                off = i * (16 * _UNROLL) + u * 16
                x = buf[pl.ds(off, 16)]
                bits = lax.bitcast_convert_type(x, I32)
                d1 = lax.shift_right_logical(bits, 19)
                plsc.addupdate_scatter(hist, [lanes * 2048 + d1], ones)
            return c

        lax.fori_loop(0, VPC // _UNROLL, ph, 0)
        tb, _ = _threshold_find(hist, 2048, 64)

        def c0(i, c):
            for u in range(_UNROLL):
                off = i * (16 * _UNROLL) + u * 16
                x = buf[pl.ds(off, 16)]
                bits = lax.bitcast_convert_type(x, I32)
                c = compact_fn(bits, base + off + lanes, tb, c)
            return c

        carry = lax.fori_loop(0, VPC // _UNROLL, c0, (zero, zero))

        for ci in range(1, NCH):
            pltpu.sync_copy(mm.at[b, pl.ds(base + ci * CH, CH)], buf)
            cbase = base + ci * CH

            def fb(i, c, _cbase=cbase):
                for u in range(_UNROLL):
                    off = i * (16 * _UNROLL) + u * 16
                    x = buf[pl.ds(off, 16)]
                    bits = lax.bitcast_convert_type(x, I32)
                    d1 = lax.shift_right_logical(bits, 19)
                    plsc.addupdate_scatter(hist, [lanes * 2048 + d1], ones)
                    c = compact_fn(bits, _cbase + off + lanes, tb, c)
                return c

            carry = lax.fori_loop(0, VPC // _UNROLL, fb, carry)

        n_st, n_ge = carry
        b1, cnt1 = _threshold_find(hist, 2048, 128)
        tgt2 = 128 - cnt1
        ok = jnp.logical_and(n_st == n_ge, b1 >= tb)

    _zero_vmem(hist, 32768)

    @pl.when(ok)
    def _():
        def f2(x, gidx, valid, c):
            msk = jnp.logical_and(
                valid, lax.shift_right_logical(x, 19) == b1)
            d2 = jnp.bitwise_and(lax.shift_right_logical(x, 8), 0x7FF)
            plsc.addupdate_scatter(hist, [lanes * 2048 + d2], ones,
                                   mask=msk)
            return c

        _scan_coll(colb, coli, n_st, f2, 0)

    @pl.when(jnp.logical_not(ok))
    def _():
        def p2(bits, gidx, c):
            msk = lax.shift_right_logical(bits, 19) == b1
            d2 = jnp.bitwise_and(lax.shift_right_logical(bits, 8), 0x7FF)
            plsc.addupdate_scatter(hist, [lanes * 2048 + d2], ones,
                                   mask=msk)
            return c

        _scan_chunks(mm, b, base, buf, p2, 0, unroll=1)

    b2, cnt2 = _threshold_find(hist, 2048, tgt2)
    tgt3 = tgt2 - cnt2
    p20 = b1 * 2048 + b2

    _zero_vmem(hist, 4096)

    @pl.when(ok)
    def _():
        def f3(x, gidx, valid, c):
            msk = jnp.logical_and(
                valid, lax.shift_right_logical(x, 8) == p20)
            d3 = jnp.bitwise_and(x, 0xFF)
            plsc.addupdate_scatter(hist, [lanes * 256 + d3], ones,
                                   mask=msk)
            return c

        _scan_coll(colb, coli, n_st, f3, 0)

    @pl.when(jnp.logical_not(ok))
    def _():
        def p3(bits, gidx, c):
            msk = lax.shift_right_logical(bits, 8) == p20
            d3 = jnp.bitwise_and(bits, 0xFF)
            plsc.addupdate_scatter(hist, [lanes * 256 + d3], ones,
                                   mask=msk)
            return c

        _scan_chunks(mm, b, base, buf, p3, 0, unroll=1)

    b3, _ = _threshold_find(hist, 256, tgt3)
    thr = p20 * 256 + b3

    neg1 = jnp.full((16,), -1, I32)
    for v in range(10):
        cand2[pl.ds(v * 16, 16)] = neg1
        cand2[pl.ds(160 + v * 16, 16)] = 0x7FF00000 + v * 16 + lanes

    def p4(bits, gidx, valid, c):
        gt_off, eq_off = c
        mg = jnp.logical_and(valid, bits > thr)
        plsc.store_compressed(cand2.at[pl.ds(gt_off, 16)], bits, mask=mg)
        plsc.store_compressed(cand2.at[pl.ds(160 + gt_off, 16)], gidx,
                              mask=mg)
        gt_off = gt_off + jnp.sum(mg.astype(I32))
        cap = jnp.full((16,), eq_off, I32) < 128
        me = jnp.logical_and(jnp.logical_and(valid, bits == thr), cap)
        plsc.store_compressed(eqi.at[pl.ds(eq_off, 16)], gidx, mask=me)
        eq_off = eq_off + jnp.sum(me.astype(I32))
        return gt_off, eq_off

    @pl.when(ok)
    def _():
        gt_off, _eq = _scan_coll(colb, coli, n_st, p4, (zero, zero))
        smem[0] = gt_off

    @pl.when(jnp.logical_not(ok))
    def _():
        def p4f(bits, gidx, c):
            tv = jnp.full((16,), 1, I32) > 0
            return p4(bits, gidx, tv, c)

        gt_off, _eq = _scan_chunks(mm, b, base, buf, p4f, (zero, zero),
                                   unroll=1)
        smem[0] = gt_off

    count_gt = smem[0]
    need_eq = 128 - count_gt

    thr_vec = jnp.full((16,), thr, I32)
    for v in range(8):
        @pl.when(v * 16 < need_eq)
        def _(v=v):
            kk = need_eq - v * 16
            msk = lanes < kk
            ev = eqi[pl.ds(v * 16, 16)]
            plsc.store_compressed(cand2.at[pl.ds(count_gt + v * 16, 16)],
                                  thr_vec, mask=msk)
            plsc.store_compressed(
                cand2.at[pl.ds(160 + count_gt + v * 16, 16)], ev, mask=msk)

    zf = jnp.zeros((16,), jnp.float32)

    def zr(i, c):
        rowbuf[pl.ds(i * 16, 16)] = zf
        return c

    lax.fori_loop(0, 80, zr, 0)

    pltpu.sync_copy(cand2, cands_s.at[pl.ds(grp * 1280 + mem * 320, 320)])
    plsc.subcore_barrier()

    pltpu.sync_copy(cands_s.at[pl.ds(grp * 1280, 1280)], allc)

    def rank_body(v, c):
        ob = allc[pl.ds(mem * 320 + v * 16, 16)]
        oi = allc[pl.ds(mem * 320 + 160 + v * 16, 16)]

        def jt_body(jt, r):
            def w_body(w, r2):
                cb = allc[pl.ds(jt * 320 + w * 16, 16)]
                cv = allc[pl.ds(jt * 320 + 160 + w * 16, 16)]
                for k in range(16):
                    perm = jnp.bitwise_and(lanes + k, 15)
                    rb = cb.at[perm].get(mode="promise_in_bounds",
                                         unique_indices=True)
                    ri = cv.at[perm].get(mode="promise_in_bounds",
                                         unique_indices=True)
                    better = jnp.logical_or(
                        rb > ob,
                        jnp.logical_and(rb == ob, ri < oi))
                    r2 = r2 + better.astype(I32)
                return r2

            return lax.fori_loop(0, 8, w_body, r)

        r = lax.fori_loop(0, 4, jt_body, jnp.zeros((16,), I32))

        safe_i = jnp.where(ob >= 0, oi, 0)
        ii = lax.shift_right_logical(safe_i, 10)
        jj = jnp.bitwise_and(safe_i, 1023)
        tgt = jnp.where(r < 128, r,
                        128 + jnp.bitwise_and(v * 16 + lanes, 31))
        t8 = tgt * 8
        for d in range(3):
            dcol = jnp.full((16,), d, I32)
            g0 = plsc.load_gather(pts0v, [ii * 3 + d])
            plsc.store_scatter(rowbuf, [t8 + d], g0)
            g1 = plsc.load_gather(pts1v, [jj * 3 + d])
            plsc.store_scatter(rowbuf, [t8 + 4 + d], g1)
        return c

    lax.fori_loop(0, 8, rank_body, 0)

    slot = (grp * 4 + mem) * 1280
    pltpu.sync_copy(rowbuf, rows_s.at[pl.ds(slot, 1280)])
    plsc.subcore_barrier()

    @pl.when(mem == 0)
    def _():
        for t in range(1, 4):
            pltpu.sync_copy(rows_s.at[pl.ds((grp * 4 + t) * 1280, 1280)],
                            mrg)

            def madd(i, c):
                rowbuf[pl.ds(i * 16, 16)] = (rowbuf[pl.ds(i * 16, 16)]
                                             + mrg[pl.ds(i * 16, 16)])
                return c

            lax.fori_loop(0, 64, madd, 0)
        pltpu.sync_copy(rowbuf.at[pl.ds(0, 1024)], out.at[b])


def _sc_topk_anchors(mm, pts0f, pts1f):
    mesh = plsc.VectorSubcoreMesh(core_axis_name="c", subcore_axis_name="s")
    fn = pl.kernel(
        _sc_body,
        out_type=jax.ShapeDtypeStruct((N, A * 8), jnp.float32),
        mesh=mesh,
        scratch_types=[
            pltpu.VMEM((CH,), jnp.float32),
            pltpu.VMEM((32768,), I32),
            pltpu.VMEM((32832,), I32),
            pltpu.VMEM((32832,), I32),
            pltpu.VMEM((160,), I32),
            pltpu.VMEM((320,), I32),
            pltpu.VMEM((1280,), I32),
            pltpu.VMEM((1280,), jnp.float32),
            pltpu.VMEM((1280,), jnp.float32),
            pltpu.VMEM((3 * L,), jnp.float32),
            pltpu.VMEM((3 * L,), jnp.float32),
            pltpu.SMEM((8,), I32),
            pltpu.VMEM_SHARED((5120,), I32),
            pltpu.VMEM_SHARED((16 * 1280,), jnp.float32),
        ],
        compiler_params=pltpu.CompilerParams(needs_layout_passes=False),
    )
    return fn(mm, pts0f, pts1f)


def _dense_body(p_ref, a_ref, o0_ref, o1_ref):
    p = p_ref[0]
    a = a_ref[0]
    for side, o_ref in ((0, o0_ref), (1, o1_ref)):
        diffs = []
        for c in range(3):
            prow = p[4 * side + c: 4 * side + c + 1, :]
            acol = a[:, 4 * side + c: 4 * side + c + 1]
            diffs.append(prow - acol)
        dist = jnp.sqrt(diffs[0] * diffs[0] + diffs[1] * diffs[1]
                        + diffs[2] * diffs[2])
        feats = diffs + [dist]
        for d, f in enumerate(feats):
            norm = jnp.sum(jnp.abs(f), axis=0, keepdims=True)
            o_ref[0, d * A:(d + 1) * A, :] = f / norm


def _dense_call(P, anchors):
    return pl.pallas_call(
        _dense_body,
        grid=(N,),
        in_specs=[
            pl.BlockSpec((1, 8, L), lambda b: (b, 0, 0)),
            pl.BlockSpec((1, A, 8), lambda b: (b, 0, 0)),
        ],
        out_specs=[
            pl.BlockSpec((1, 4 * A, L), lambda b: (b, 0, 0)),
            pl.BlockSpec((1, 4 * A, L), lambda b: (b, 0, 0)),
        ],
        out_shape=[
            jax.ShapeDtypeStruct((N, 4 * A, L), jnp.float32),
            jax.ShapeDtypeStruct((N, 4 * A, L), jnp.float32),
        ],
    )(P, anchors)


def kernel(match_mask, pts_3d0, pts_3d1, K0, K1, non_epipolar):
    mm = match_mask.reshape(N, FLAT)
    anchors = _sc_topk_anchors(mm, pts_3d0.reshape(N, 3 * L),
                               pts_3d1.reshape(N, 3 * L))
    anchors = anchors.reshape(N, A, 8)
    z2 = jnp.zeros((N, 1, L), jnp.float32)
    P = jnp.concatenate(
        [pts_3d0.transpose(0, 2, 1), z2, pts_3d1.transpose(0, 2, 1), z2],
        axis=1)
    out0, out1 = _dense_call(P, anchors)
    return (out0.reshape(N, 4 * A, H, W), out1.reshape(N, 4 * A, H, W))


# speedup vs baseline: 4.9680x; 1.0211x over previous
import functools

import jax
import jax.numpy as jnp
from jax import lax
from jax.experimental import pallas as pl
from jax.experimental.pallas import tpu as pltpu
from jax.experimental.pallas import tpu_sc as plsc

N, L, S = 8, 1024, 1024
H, W = 32, 32
A = 128
FLAT = L * S
QUART = FLAT // 4
CH = 16384
NCH = QUART // CH
VPC = CH // 16

I32 = jnp.int32


def _lanes():
    return lax.iota(I32, 16)


def _threshold_find(hist, nb, tgt):
    ng = nb // 16
    lanes = _lanes()
    zero = jnp.zeros((), I32)

    def body(k, carry):
        found, bkt, cnt, acc = carry
        g = ng - 1 - k
        tot = hist[pl.ds(g * 16, 16)]
        for ln in range(1, 16):
            tot = tot + hist[pl.ds(ln * nb + g * 16, 16)]
        rev = lax.rev(tot, (0,))
        csum = plsc.cumsum(rev)
        cross = (acc + csum) >= tgt
        has = jnp.sum(cross.astype(I32)) > 0
        kv = plsc.all_reduce_ffs(cross)
        ks = jnp.max(kv)
        csel = jnp.sum(jnp.where(lanes == ks, csum, 0).astype(I32))
        rsel = jnp.sum(jnp.where(lanes == ks, rev, 0).astype(I32))
        hit = jnp.logical_and(found == 0, has)
        bkt = jnp.where(hit, g * 16 + 15 - ks, bkt)
        cnt = jnp.where(hit, acc + csel - rsel, cnt)
        found = jnp.where(has, jnp.ones((), I32), found)
        acc = jnp.where(found > 0, acc, acc + jnp.sum(tot))
        return found, bkt, cnt, acc

    _, bkt, cnt, _ = lax.fori_loop(0, ng, body,
                                   (zero, zero, zero, zero))
    return bkt, cnt


def _zero_vmem(ref, n):
    z = jnp.zeros((16,), I32)

    def zb(i, c):
        for u in range(8):
            ref[pl.ds(i * 128 + u * 16, 16)] = z
        return c

    lax.fori_loop(0, n // 128, zb, 0)


_UNROLL = 4


def _scan_chunks(mm, b, base, buf, fn, carry, unroll=_UNROLL):
    lanes = _lanes()
    for ci in range(NCH):
        pltpu.sync_copy(mm.at[b, pl.ds(base + ci * CH, CH)], buf)
        cbase = base + ci * CH

        def vbody(i, c, _cbase=cbase):
            for u in range(unroll):
                off = i * (16 * unroll) + u * 16
                x = buf[pl.ds(off, 16)]
                bits = lax.bitcast_convert_type(x, I32)
                gidx = _cbase + off + lanes
                c = fn(bits, gidx, c)
            return c

        carry = lax.fori_loop(0, VPC // unroll, vbody, carry)
    return carry


def _scan_coll(colb, coli, n, fn, carry, unroll=_UNROLL):
    lanes = _lanes()

    def vbody(i, c):
        for u in range(unroll):
            off = i * (16 * unroll) + u * 16
            x = colb[pl.ds(off, 16)]
            gidx = coli[pl.ds(off, 16)]
            valid = (off + lanes) < n
            c = fn(x, gidx, valid, c)
        return c

    nv = (n + 16 * unroll - 1) // (16 * unroll)
    return lax.fori_loop(0, nv, vbody, carry)


_CAP = 32752


def _sc_body(mm, pts0, pts1, out, buf, hist, colb, coli, eqi, cand2, allc,
             rowbuf, mrg, pts0v, pts1v, smem, cands_s, rows_s):
    lanes = _lanes()
    ones = jnp.full((16,), 1, I32)
    zero = jnp.zeros((), I32)
    cc = lax.axis_index("c")
    ss = lax.axis_index("s")
    grp = ss // 4
    mem = ss % 4
    b = cc * 4 + grp
    base = mem * QUART

    pltpu.sync_copy(pts0.at[b], pts0v)
    pltpu.sync_copy(pts1.at[b], pts1v)

    with jax.named_scope("p1_fused"):
        _zero_vmem(hist, 32768)

        def compact_fn(bits, gidx, tbits, c):
            n_st, n_ge = c
            mge = bits >= tbits
            mst = jnp.logical_and(mge, jnp.full((16,), n_st, I32) < _CAP)
            plsc.store_compressed(colb.at[pl.ds(n_st, 16)], bits, mask=mst)
            plsc.store_compressed(coli.at[pl.ds(n_st, 16)], gidx, mask=mst)
            n_st = n_st + jnp.sum(mst.astype(I32))
            n_ge = n_ge + jnp.sum(mge.astype(I32))
            return n_st, n_ge

        pltpu.sync_copy(mm.at[b, pl.ds(base, CH)], buf)

        def ph(i, c):
            for u in range(_UNROLL):
                off = i * (16 * _UNROLL) + u * 16
                x = buf[pl.ds(off, 16)]
                bits = lax.bitcast_convert_type(x, I32)
                d1 = lax.shift_right_logical(bits, 19)
                plsc.addupdate_scatter(hist, [lanes * 2048 + d1], ones)
            return c

        lax.fori_loop(0, VPC // _UNROLL, ph, 0)
        tb, _ = _threshold_find(hist, 2048, 64)
        tbits = tb * 524288

        def c0(i, c):
            for u in range(_UNROLL):
                off = i * (16 * _UNROLL) + u * 16
                x = buf[pl.ds(off, 16)]
                bits = lax.bitcast_convert_type(x, I32)
                c = compact_fn(bits, base + off + lanes, tbits, c)
            return c

        carry = lax.fori_loop(0, VPC // _UNROLL, c0, (zero, zero))

        for ci in range(1, NCH):
            pltpu.sync_copy(mm.at[b, pl.ds(base + ci * CH, CH)], buf)
            cbase = base + ci * CH

            def fb(i, c, _cbase=cbase):
                for u in range(_UNROLL):
                    off = i * (16 * _UNROLL) + u * 16
                    x = buf[pl.ds(off, 16)]
                    bits = lax.bitcast_convert_type(x, I32)
                    c = compact_fn(bits, _cbase + off + lanes, tbits, c)
                return c

            carry = lax.fori_loop(0, VPC // _UNROLL, fb, carry)

        n_st, n_ge = carry
        ok = jnp.logical_and(n_st == n_ge, n_ge >= 128)

    _zero_vmem(hist, 32768)

    @pl.when(ok)
    def _():
        def f1(x, gidx, valid, c):
            d1 = lax.shift_right_logical(x, 19)
            plsc.addupdate_scatter(hist, [lanes * 2048 + d1], ones,
                                   mask=valid)
            return c

        _scan_coll(colb, coli, n_st, f1, 0)

    @pl.when(jnp.logical_not(ok))
    def _():
        def p1f(bits, gidx, c):
            d1 = lax.shift_right_logical(bits, 19)
            plsc.addupdate_scatter(hist, [lanes * 2048 + d1], ones)
            return c

        _scan_chunks(mm, b, base, buf, p1f, 0, unroll=1)

    b1, cnt1 = _threshold_find(hist, 2048, 128)
    tgt2 = 128 - cnt1

    _zero_vmem(hist, 32768)

    @pl.when(ok)
    def _():
        def f2(x, gidx, valid, c):
            msk = jnp.logical_and(
                valid, lax.shift_right_logical(x, 19) == b1)
            d2 = jnp.bitwise_and(lax.shift_right_logical(x, 8), 0x7FF)
            plsc.addupdate_scatter(hist, [lanes * 2048 + d2], ones,
                                   mask=msk)
            return c

        _scan_coll(colb, coli, n_st, f2, 0)

    @pl.when(jnp.logical_not(ok))
    def _():
        def p2(bits, gidx, c):
            msk = lax.shift_right_logical(bits, 19) == b1
            d2 = jnp.bitwise_and(lax.shift_right_logical(bits, 8), 0x7FF)
            plsc.addupdate_scatter(hist, [lanes * 2048 + d2], ones,
                                   mask=msk)
            return c

        _scan_chunks(mm, b, base, buf, p2, 0, unroll=1)

    b2, cnt2 = _threshold_find(hist, 2048, tgt2)
    tgt3 = tgt2 - cnt2
    p20 = b1 * 2048 + b2

    _zero_vmem(hist, 4096)

    @pl.when(ok)
    def _():
        def f3(x, gidx, valid, c):
            msk = jnp.logical_and(
                valid, lax.shift_right_logical(x, 8) == p20)
            d3 = jnp.bitwise_and(x, 0xFF)
            plsc.addupdate_scatter(hist, [lanes * 256 + d3], ones,
                                   mask=msk)
            return c

        _scan_coll(colb, coli, n_st, f3, 0)

    @pl.when(jnp.logical_not(ok))
    def _():
        def p3(bits, gidx, c):
            msk = lax.shift_right_logical(bits, 8) == p20
            d3 = jnp.bitwise_and(bits, 0xFF)
            plsc.addupdate_scatter(hist, [lanes * 256 + d3], ones,
                                   mask=msk)
            return c

        _scan_chunks(mm, b, base, buf, p3, 0, unroll=1)

    b3, _ = _threshold_find(hist, 256, tgt3)
    thr = p20 * 256 + b3

    neg1 = jnp.full((16,), -1, I32)
    for v in range(10):
        cand2[pl.ds(v * 16, 16)] = neg1
        cand2[pl.ds(160 + v * 16, 16)] = 0x7FF00000 + v * 16 + lanes

    def p4(bits, gidx, valid, c):
        gt_off, eq_off = c
        mg = jnp.logical_and(valid, bits > thr)
        plsc.store_compressed(cand2.at[pl.ds(gt_off, 16)], bits, mask=mg)
        plsc.store_compressed(cand2.at[pl.ds(160 + gt_off, 16)], gidx,
                              mask=mg)
        gt_off = gt_off + jnp.sum(mg.astype(I32))
        cap = jnp.full((16,), eq_off, I32) < 128
        me = jnp.logical_and(jnp.logical_and(valid, bits == thr), cap)
        plsc.store_compressed(eqi.at[pl.ds(eq_off, 16)], gidx, mask=me)
        eq_off = eq_off + jnp.sum(me.astype(I32))
        return gt_off, eq_off

    @pl.when(ok)
    def _():
        gt_off, _eq = _scan_coll(colb, coli, n_st, p4, (zero, zero))
        smem[0] = gt_off

    @pl.when(jnp.logical_not(ok))
    def _():
        def p4f(bits, gidx, c):
            tv = jnp.full((16,), 1, I32) > 0
            return p4(bits, gidx, tv, c)

        gt_off, _eq = _scan_chunks(mm, b, base, buf, p4f, (zero, zero),
                                   unroll=1)
        smem[0] = gt_off

    count_gt = smem[0]
    need_eq = 128 - count_gt

    thr_vec = jnp.full((16,), thr, I32)
    for v in range(8):
        @pl.when(v * 16 < need_eq)
        def _(v=v):
            kk = need_eq - v * 16
            msk = lanes < kk
            ev = eqi[pl.ds(v * 16, 16)]
            plsc.store_compressed(cand2.at[pl.ds(count_gt + v * 16, 16)],
                                  thr_vec, mask=msk)
            plsc.store_compressed(
                cand2.at[pl.ds(160 + count_gt + v * 16, 16)], ev, mask=msk)

    zf = jnp.zeros((16,), jnp.float32)

    def zr(i, c):
        rowbuf[pl.ds(i * 16, 16)] = zf
        return c

    lax.fori_loop(0, 80, zr, 0)

    pltpu.sync_copy(cand2, cands_s.at[pl.ds(grp * 1280 + mem * 320, 320)])
    plsc.subcore_barrier()

    pltpu.sync_copy(cands_s.at[pl.ds(grp * 1280, 1280)], allc)

    def rank_body(v, c):
        ob = allc[pl.ds(mem * 320 + v * 16, 16)]
        oi = allc[pl.ds(mem * 320 + 160 + v * 16, 16)]

        def jt_body(jt, r):
            def w_body(w, r2):
                cb = allc[pl.ds(jt * 320 + w * 16, 16)]
                cv = allc[pl.ds(jt * 320 + 160 + w * 16, 16)]
                for k in range(16):
                    perm = jnp.bitwise_and(lanes + k, 15)
                    rb = cb.at[perm].get(mode="promise_in_bounds",
                                         unique_indices=True)
                    ri = cv.at[perm].get(mode="promise_in_bounds",
                                         unique_indices=True)
                    better = jnp.logical_or(
                        rb > ob,
                        jnp.logical_and(rb == ob, ri < oi))
                    r2 = r2 + better.astype(I32)
                return r2

            return lax.fori_loop(0, 8, w_body, r)

        r = lax.fori_loop(0, 4, jt_body, jnp.zeros((16,), I32))

        safe_i = jnp.where(ob >= 0, oi, 0)
        ii = lax.shift_right_logical(safe_i, 10)
        jj = jnp.bitwise_and(safe_i, 1023)
        tgt = jnp.where(r < 128, r,
                        128 + jnp.bitwise_and(v * 16 + lanes, 31))
        t8 = tgt * 8
        for d in range(3):
            dcol = jnp.full((16,), d, I32)
            g0 = plsc.load_gather(pts0v, [ii * 3 + d])
            plsc.store_scatter(rowbuf, [t8 + d], g0)
            g1 = plsc.load_gather(pts1v, [jj * 3 + d])
            plsc.store_scatter(rowbuf, [t8 + 4 + d], g1)
        return c

    lax.fori_loop(0, 8, rank_body, 0)

    slot = (grp * 4 + mem) * 1280
    pltpu.sync_copy(rowbuf, rows_s.at[pl.ds(slot, 1280)])
    plsc.subcore_barrier()

    @pl.when(mem == 0)
    def _():
        for t in range(1, 4):
            pltpu.sync_copy(rows_s.at[pl.ds((grp * 4 + t) * 1280, 1280)],
                            mrg)

            def madd(i, c):
                rowbuf[pl.ds(i * 16, 16)] = (rowbuf[pl.ds(i * 16, 16)]
                                             + mrg[pl.ds(i * 16, 16)])
                return c

            lax.fori_loop(0, 64, madd, 0)
        pltpu.sync_copy(rowbuf.at[pl.ds(0, 1024)], out.at[b])


def _sc_topk_anchors(mm, pts0f, pts1f):
    mesh = plsc.VectorSubcoreMesh(core_axis_name="c", subcore_axis_name="s")
    fn = pl.kernel(
        _sc_body,
        out_type=jax.ShapeDtypeStruct((N, A * 8), jnp.float32),
        mesh=mesh,
        scratch_types=[
            pltpu.VMEM((CH,), jnp.float32),
            pltpu.VMEM((32768,), I32),
            pltpu.VMEM((32832,), I32),
            pltpu.VMEM((32832,), I32),
            pltpu.VMEM((160,), I32),
            pltpu.VMEM((320,), I32),
            pltpu.VMEM((1280,), I32),
            pltpu.VMEM((1280,), jnp.float32),
            pltpu.VMEM((1280,), jnp.float32),
            pltpu.VMEM((3 * L,), jnp.float32),
            pltpu.VMEM((3 * L,), jnp.float32),
            pltpu.SMEM((8,), I32),
            pltpu.VMEM_SHARED((5120,), I32),
            pltpu.VMEM_SHARED((16 * 1280,), jnp.float32),
        ],
        compiler_params=pltpu.CompilerParams(needs_layout_passes=False),
    )
    return fn(mm, pts0f, pts1f)


def _dense_body(p_ref, a_ref, o0_ref, o1_ref):
    p = p_ref[0]
    a = a_ref[0]
    for side, o_ref in ((0, o0_ref), (1, o1_ref)):
        diffs = []
        for c in range(3):
            prow = p[4 * side + c: 4 * side + c + 1, :]
            acol = a[:, 4 * side + c: 4 * side + c + 1]
            diffs.append(prow - acol)
        dist = jnp.sqrt(diffs[0] * diffs[0] + diffs[1] * diffs[1]
                        + diffs[2] * diffs[2])
        feats = diffs + [dist]
        for d, f in enumerate(feats):
            norm = jnp.sum(jnp.abs(f), axis=0, keepdims=True)
            o_ref[0, d * A:(d + 1) * A, :] = f / norm


def _dense_call(P, anchors):
    return pl.pallas_call(
        _dense_body,
        grid=(N,),
        in_specs=[
            pl.BlockSpec((1, 8, L), lambda b: (b, 0, 0)),
            pl.BlockSpec((1, A, 8), lambda b: (b, 0, 0)),
        ],
        out_specs=[
            pl.BlockSpec((1, 4 * A, L), lambda b: (b, 0, 0)),
            pl.BlockSpec((1, 4 * A, L), lambda b: (b, 0, 0)),
        ],
        out_shape=[
            jax.ShapeDtypeStruct((N, 4 * A, L), jnp.float32),
            jax.ShapeDtypeStruct((N, 4 * A, L), jnp.float32),
        ],
    )(P, anchors)


def kernel(match_mask, pts_3d0, pts_3d1, K0, K1, non_epipolar):
    mm = match_mask.reshape(N, FLAT)
    anchors = _sc_topk_anchors(mm, pts_3d0.reshape(N, 3 * L),
                               pts_3d1.reshape(N, 3 * L))
    anchors = anchors.reshape(N, A, 8)
    z2 = jnp.zeros((N, 1, L), jnp.float32)
    P = jnp.concatenate(
        [pts_3d0.transpose(0, 2, 1), z2, pts_3d1.transpose(0, 2, 1), z2],
        axis=1)
    out0, out1 = _dense_call(P, anchors)
    return (out0.reshape(N, 4 * A, H, W), out1.reshape(N, 4 * A, H, W))


# speedup vs baseline: 7.1031x; 1.4298x over previous
import functools

import jax
import jax.numpy as jnp
from jax import lax
from jax.experimental import pallas as pl
from jax.experimental.pallas import tpu as pltpu
from jax.experimental.pallas import tpu_sc as plsc

N, L, S = 8, 1024, 1024
H, W = 32, 32
A = 128
FLAT = L * S
QUART = FLAT // 4
CH = 16384
NCH = QUART // CH
VPC = CH // 16

I32 = jnp.int32


def _lanes():
    return lax.iota(I32, 16)


def _threshold_find(hist, nb, tgt):
    ng = nb // 16
    lanes = _lanes()
    zero = jnp.zeros((), I32)

    def body(k, carry):
        found, bkt, cnt, acc = carry
        g = ng - 1 - k
        tot = hist[pl.ds(g * 16, 16)]
        for ln in range(1, 16):
            tot = tot + hist[pl.ds(ln * nb + g * 16, 16)]
        rev = lax.rev(tot, (0,))
        csum = plsc.cumsum(rev)
        cross = (acc + csum) >= tgt
        has = jnp.sum(cross.astype(I32)) > 0
        kv = plsc.all_reduce_ffs(cross)
        ks = jnp.max(kv)
        csel = jnp.sum(jnp.where(lanes == ks, csum, 0).astype(I32))
        rsel = jnp.sum(jnp.where(lanes == ks, rev, 0).astype(I32))
        hit = jnp.logical_and(found == 0, has)
        bkt = jnp.where(hit, g * 16 + 15 - ks, bkt)
        cnt = jnp.where(hit, acc + csel - rsel, cnt)
        found = jnp.where(has, jnp.ones((), I32), found)
        acc = jnp.where(found > 0, acc, acc + jnp.sum(tot))
        return found, bkt, cnt, acc

    _, bkt, cnt, _ = lax.fori_loop(0, ng, body,
                                   (zero, zero, zero, zero))
    return bkt, cnt


def _zero_vmem(ref, n):
    z = jnp.zeros((16,), I32)

    def zb(i, c):
        for u in range(8):
            ref[pl.ds(i * 128 + u * 16, 16)] = z
        return c

    lax.fori_loop(0, n // 128, zb, 0)


_UNROLL = 4


def _scan_chunks(mm, b, base, buf, fn, carry, unroll=_UNROLL):
    lanes = _lanes()
    for ci in range(NCH):
        pltpu.sync_copy(mm.at[b, pl.ds(base + ci * CH, CH)], buf)
        cbase = base + ci * CH

        def vbody(i, c, _cbase=cbase):
            for u in range(unroll):
                off = i * (16 * unroll) + u * 16
                x = buf[pl.ds(off, 16)]
                bits = lax.bitcast_convert_type(x, I32)
                gidx = _cbase + off + lanes
                c = fn(bits, gidx, c)
            return c

        carry = lax.fori_loop(0, VPC // unroll, vbody, carry)
    return carry


def _scan_coll(colb, coli, n, fn, carry, unroll=_UNROLL):
    lanes = _lanes()

    def vbody(i, c):
        for u in range(unroll):
            off = i * (16 * unroll) + u * 16
            x = colb[pl.ds(off, 16)]
            gidx = coli[pl.ds(off, 16)]
            valid = (off + lanes) < n
            c = fn(x, gidx, valid, c)
        return c

    nv = (n + 16 * unroll - 1) // (16 * unroll)
    return lax.fori_loop(0, nv, vbody, carry)


_CAP = 16432


def _sc_body(mm, pts0, pts1, out, buf, hist, colb, coli, eqi, cand2, allc,
             rowbuf, mrg, pts0v, pts1v, smem, cands_s, rows_s):
    lanes = _lanes()
    ones = jnp.full((16,), 1, I32)
    zero = jnp.zeros((), I32)
    cc = lax.axis_index("c")
    ss = lax.axis_index("s")
    grp = ss // 4
    mem = ss % 4
    b = cc * 4 + grp
    base = mem * QUART

    pltpu.sync_copy(pts0.at[b], pts0v)
    pltpu.sync_copy(pts1.at[b], pts1v)

    with jax.named_scope("p1_fused"):
        _zero_vmem(hist, 32768)

        def compact_fn(bits, gidx, capv, c):
            n_st, ngev = c
            mge = bits >= tbits
            mst = jnp.logical_and(mge, capv)
            plsc.store_compressed(colb.at[pl.ds(n_st, 16)], bits, mask=mst)
            plsc.store_compressed(coli.at[pl.ds(n_st, 16)], gidx, mask=mst)
            n_st = n_st + plsc.all_reduce_population_count(mst)[0]
            ngev = ngev + mge.astype(I32)
            return n_st, ngev

        pltpu.sync_copy(mm.at[b, pl.ds(base, CH)], buf)

        def ph(i, c):
            for u in range(_UNROLL):
                off = i * (16 * _UNROLL) + u * 16
                x = buf[pl.ds(off, 16)]
                bits = lax.bitcast_convert_type(x, I32)
                d1 = lax.shift_right_logical(bits, 19)
                plsc.addupdate_scatter(hist, [lanes * 2048 + d1], ones)
            return c

        lax.fori_loop(0, VPC // _UNROLL, ph, 0)
        tb, _ = _threshold_find(hist, 2048, 64)
        tbits = tb * 524288

        zvec = jnp.zeros((16,), I32)
        carry = (zero, zvec)
        for ci in range(NCH):
            if ci > 0:
                pltpu.sync_copy(mm.at[b, pl.ds(base + ci * CH, CH)], buf)
            cbase = base + ci * CH
            capv = jnp.full((16,), carry[0], I32) < _CAP

            def fb(i, c, _cbase=cbase, _capv=capv):
                for u in range(_UNROLL):
                    off = i * (16 * _UNROLL) + u * 16
                    x = buf[pl.ds(off, 16)]
                    bits = lax.bitcast_convert_type(x, I32)
                    c = compact_fn(bits, _cbase + off + lanes, _capv, c)
                return c

            carry = lax.fori_loop(0, VPC // _UNROLL, fb, carry)

        n_st, ngev = carry
        n_ge = jnp.sum(ngev)
        ok = jnp.logical_and(n_st == n_ge, n_ge >= 128)

    _zero_vmem(hist, 32768)

    @pl.when(ok)
    def _():
        def f1(x, gidx, valid, c):
            d1 = lax.shift_right_logical(x, 19)
            plsc.addupdate_scatter(hist, [lanes * 2048 + d1], ones,
                                   mask=valid)
            return c

        _scan_coll(colb, coli, n_st, f1, 0)

    @pl.when(jnp.logical_not(ok))
    def _():
        def p1f(bits, gidx, c):
            d1 = lax.shift_right_logical(bits, 19)
            plsc.addupdate_scatter(hist, [lanes * 2048 + d1], ones)
            return c

        _scan_chunks(mm, b, base, buf, p1f, 0, unroll=1)

    b1, cnt1 = _threshold_find(hist, 2048, 128)
    tgt2 = 128 - cnt1

    _zero_vmem(hist, 32768)

    @pl.when(ok)
    def _():
        def f2(x, gidx, valid, c):
            msk = jnp.logical_and(
                valid, lax.shift_right_logical(x, 19) == b1)
            d2 = jnp.bitwise_and(lax.shift_right_logical(x, 8), 0x7FF)
            plsc.addupdate_scatter(hist, [lanes * 2048 + d2], ones,
                                   mask=msk)
            return c

        _scan_coll(colb, coli, n_st, f2, 0)

    @pl.when(jnp.logical_not(ok))
    def _():
        def p2(bits, gidx, c):
            msk = lax.shift_right_logical(bits, 19) == b1
            d2 = jnp.bitwise_and(lax.shift_right_logical(bits, 8), 0x7FF)
            plsc.addupdate_scatter(hist, [lanes * 2048 + d2], ones,
                                   mask=msk)
            return c

        _scan_chunks(mm, b, base, buf, p2, 0, unroll=1)

    b2, cnt2 = _threshold_find(hist, 2048, tgt2)
    tgt3 = tgt2 - cnt2
    p20 = b1 * 2048 + b2

    _zero_vmem(hist, 4096)

    @pl.when(ok)
    def _():
        def f3(x, gidx, valid, c):
            msk = jnp.logical_and(
                valid, lax.shift_right_logical(x, 8) == p20)
            d3 = jnp.bitwise_and(x, 0xFF)
            plsc.addupdate_scatter(hist, [lanes * 256 + d3], ones,
                                   mask=msk)
            return c

        _scan_coll(colb, coli, n_st, f3, 0)

    @pl.when(jnp.logical_not(ok))
    def _():
        def p3(bits, gidx, c):
            msk = lax.shift_right_logical(bits, 8) == p20
            d3 = jnp.bitwise_and(bits, 0xFF)
            plsc.addupdate_scatter(hist, [lanes * 256 + d3], ones,
                                   mask=msk)
            return c

        _scan_chunks(mm, b, base, buf, p3, 0, unroll=1)

    b3, _ = _threshold_find(hist, 256, tgt3)
    thr = p20 * 256 + b3

    neg1 = jnp.full((16,), -1, I32)
    for v in range(10):
        cand2[pl.ds(v * 16, 16)] = neg1
        cand2[pl.ds(160 + v * 16, 16)] = 0x7FF00000 + v * 16 + lanes

    def p4(bits, gidx, valid, c):
        gt_off, eq_off = c
        mg = jnp.logical_and(valid, bits > thr)
        plsc.store_compressed(cand2.at[pl.ds(gt_off, 16)], bits, mask=mg)
        plsc.store_compressed(cand2.at[pl.ds(160 + gt_off, 16)], gidx,
                              mask=mg)
        gt_off = gt_off + jnp.sum(mg.astype(I32))
        cap = jnp.full((16,), eq_off, I32) < 128
        me = jnp.logical_and(jnp.logical_and(valid, bits == thr), cap)
        plsc.store_compressed(eqi.at[pl.ds(eq_off, 16)], gidx, mask=me)
        eq_off = eq_off + jnp.sum(me.astype(I32))
        return gt_off, eq_off

    @pl.when(ok)
    def _():
        gt_off, _eq = _scan_coll(colb, coli, n_st, p4, (zero, zero))
        smem[0] = gt_off

    @pl.when(jnp.logical_not(ok))
    def _():
        def p4f(bits, gidx, c):
            tv = jnp.full((16,), 1, I32) > 0
            return p4(bits, gidx, tv, c)

        gt_off, _eq = _scan_chunks(mm, b, base, buf, p4f, (zero, zero),
                                   unroll=1)
        smem[0] = gt_off

    count_gt = smem[0]
    need_eq = 128 - count_gt

    thr_vec = jnp.full((16,), thr, I32)
    for v in range(8):
        @pl.when(v * 16 < need_eq)
        def _(v=v):
            kk = need_eq - v * 16
            msk = lanes < kk
            ev = eqi[pl.ds(v * 16, 16)]
            plsc.store_compressed(cand2.at[pl.ds(count_gt + v * 16, 16)],
                                  thr_vec, mask=msk)
            plsc.store_compressed(
                cand2.at[pl.ds(160 + count_gt + v * 16, 16)], ev, mask=msk)

    zf = jnp.zeros((16,), jnp.float32)

    def zr(i, c):
        rowbuf[pl.ds(i * 16, 16)] = zf
        return c

    lax.fori_loop(0, 80, zr, 0)

    pltpu.sync_copy(cand2, cands_s.at[pl.ds(grp * 1280 + mem * 320, 320)])
    plsc.subcore_barrier()

    pltpu.sync_copy(cands_s.at[pl.ds(grp * 1280, 1280)], allc)

    def rank_body(v, c):
        ob = allc[pl.ds(mem * 320 + v * 16, 16)]
        oi = allc[pl.ds(mem * 320 + 160 + v * 16, 16)]

        def jt_body(jt, r):
            def w_body(w, r2):
                cb = allc[pl.ds(jt * 320 + w * 16, 16)]
                cv = allc[pl.ds(jt * 320 + 160 + w * 16, 16)]
                for k in range(16):
                    perm = jnp.bitwise_and(lanes + k, 15)
                    rb = cb.at[perm].get(mode="promise_in_bounds",
                                         unique_indices=True)
                    ri = cv.at[perm].get(mode="promise_in_bounds",
                                         unique_indices=True)
                    better = jnp.logical_or(
                        rb > ob,
                        jnp.logical_and(rb == ob, ri < oi))
                    r2 = r2 + better.astype(I32)
                return r2

            return lax.fori_loop(0, 8, w_body, r)

        r = lax.fori_loop(0, 4, jt_body, jnp.zeros((16,), I32))

        safe_i = jnp.where(ob >= 0, oi, 0)
        ii = lax.shift_right_logical(safe_i, 10)
        jj = jnp.bitwise_and(safe_i, 1023)
        tgt = jnp.where(r < 128, r,
                        128 + jnp.bitwise_and(v * 16 + lanes, 31))
        t8 = tgt * 8
        for d in range(3):
            dcol = jnp.full((16,), d, I32)
            g0 = plsc.load_gather(pts0v, [ii * 3 + d])
            plsc.store_scatter(rowbuf, [t8 + d], g0)
            g1 = plsc.load_gather(pts1v, [jj * 3 + d])
            plsc.store_scatter(rowbuf, [t8 + 4 + d], g1)
        return c

    lax.fori_loop(0, 8, rank_body, 0)

    slot = (grp * 4 + mem) * 1280
    pltpu.sync_copy(rowbuf, rows_s.at[pl.ds(slot, 1280)])
    plsc.subcore_barrier()

    @pl.when(mem == 0)
    def _():
        for t in range(1, 4):
            pltpu.sync_copy(rows_s.at[pl.ds((grp * 4 + t) * 1280, 1280)],
                            mrg)

            def madd(i, c):
                rowbuf[pl.ds(i * 16, 16)] = (rowbuf[pl.ds(i * 16, 16)]
                                             + mrg[pl.ds(i * 16, 16)])
                return c

            lax.fori_loop(0, 64, madd, 0)
        pltpu.sync_copy(rowbuf.at[pl.ds(0, 1024)], out.at[b])


def _sc_topk_anchors(mm, pts0f, pts1f):
    mesh = plsc.VectorSubcoreMesh(core_axis_name="c", subcore_axis_name="s")
    fn = pl.kernel(
        _sc_body,
        out_type=jax.ShapeDtypeStruct((N, A * 8), jnp.float32),
        mesh=mesh,
        scratch_types=[
            pltpu.VMEM((CH,), jnp.float32),
            pltpu.VMEM((32768,), I32),
            pltpu.VMEM((32832,), I32),
            pltpu.VMEM((32832,), I32),
            pltpu.VMEM((160,), I32),
            pltpu.VMEM((320,), I32),
            pltpu.VMEM((1280,), I32),
            pltpu.VMEM((1280,), jnp.float32),
            pltpu.VMEM((1280,), jnp.float32),
            pltpu.VMEM((3 * L,), jnp.float32),
            pltpu.VMEM((3 * L,), jnp.float32),
            pltpu.SMEM((8,), I32),
            pltpu.VMEM_SHARED((5120,), I32),
            pltpu.VMEM_SHARED((16 * 1280,), jnp.float32),
        ],
        compiler_params=pltpu.CompilerParams(needs_layout_passes=False),
    )
    return fn(mm, pts0f, pts1f)


def _dense_body(p_ref, a_ref, o0_ref, o1_ref):
    p = p_ref[0]
    a = a_ref[0]
    for side, o_ref in ((0, o0_ref), (1, o1_ref)):
        diffs = []
        for c in range(3):
            prow = p[4 * side + c: 4 * side + c + 1, :]
            acol = a[:, 4 * side + c: 4 * side + c + 1]
            diffs.append(prow - acol)
        dist = jnp.sqrt(diffs[0] * diffs[0] + diffs[1] * diffs[1]
                        + diffs[2] * diffs[2])
        feats = diffs + [dist]
        for d, f in enumerate(feats):
            norm = jnp.sum(jnp.abs(f), axis=0, keepdims=True)
            o_ref[0, d * A:(d + 1) * A, :] = f / norm


def _dense_call(P, anchors):
    return pl.pallas_call(
        _dense_body,
        grid=(N,),
        in_specs=[
            pl.BlockSpec((1, 8, L), lambda b: (b, 0, 0)),
            pl.BlockSpec((1, A, 8), lambda b: (b, 0, 0)),
        ],
        out_specs=[
            pl.BlockSpec((1, 4 * A, L), lambda b: (b, 0, 0)),
            pl.BlockSpec((1, 4 * A, L), lambda b: (b, 0, 0)),
        ],
        out_shape=[
            jax.ShapeDtypeStruct((N, 4 * A, L), jnp.float32),
            jax.ShapeDtypeStruct((N, 4 * A, L), jnp.float32),
        ],
    )(P, anchors)


def kernel(match_mask, pts_3d0, pts_3d1, K0, K1, non_epipolar):
    mm = match_mask.reshape(N, FLAT)
    anchors = _sc_topk_anchors(mm, pts_3d0.reshape(N, 3 * L),
                               pts_3d1.reshape(N, 3 * L))
    anchors = anchors.reshape(N, A, 8)
    z2 = jnp.zeros((N, 1, L), jnp.float32)
    P = jnp.concatenate(
        [pts_3d0.transpose(0, 2, 1), z2, pts_3d1.transpose(0, 2, 1), z2],
        axis=1)
    out0, out1 = _dense_call(P, anchors)
    return (out0.reshape(N, 4 * A, H, W), out1.reshape(N, 4 * A, H, W))


# speedup vs baseline: 7.6179x; 1.0725x over previous
import functools

import jax
import jax.numpy as jnp
from jax import lax
from jax.experimental import pallas as pl
from jax.experimental.pallas import tpu as pltpu
from jax.experimental.pallas import tpu_sc as plsc

N, L, S = 8, 1024, 1024
H, W = 32, 32
A = 128
FLAT = L * S
QUART = FLAT // 4
CH = 8192
NCH = QUART // CH
VPC = CH // 16

I32 = jnp.int32


def _lanes():
    return lax.iota(I32, 16)


def _threshold_find(hist, nb, tgt):
    ng = nb // 16
    lanes = _lanes()
    zero = jnp.zeros((), I32)

    def body(k, carry):
        found, bkt, cnt, acc = carry
        g = ng - 1 - k
        tot = hist[pl.ds(g * 16, 16)]
        for ln in range(1, 16):
            tot = tot + hist[pl.ds(ln * nb + g * 16, 16)]
        rev = lax.rev(tot, (0,))
        csum = plsc.cumsum(rev)
        cross = (acc + csum) >= tgt
        has = jnp.sum(cross.astype(I32)) > 0
        kv = plsc.all_reduce_ffs(cross)
        ks = jnp.max(kv)
        csel = jnp.sum(jnp.where(lanes == ks, csum, 0).astype(I32))
        rsel = jnp.sum(jnp.where(lanes == ks, rev, 0).astype(I32))
        hit = jnp.logical_and(found == 0, has)
        bkt = jnp.where(hit, g * 16 + 15 - ks, bkt)
        cnt = jnp.where(hit, acc + csel - rsel, cnt)
        found = jnp.where(has, jnp.ones((), I32), found)
        acc = jnp.where(found > 0, acc, acc + jnp.sum(tot))
        return found, bkt, cnt, acc

    _, bkt, cnt, _ = lax.fori_loop(0, ng, body,
                                   (zero, zero, zero, zero))
    return bkt, cnt


def _zero_vmem(ref, n):
    z = jnp.zeros((16,), I32)

    def zb(i, c):
        for u in range(8):
            ref[pl.ds(i * 128 + u * 16, 16)] = z
        return c

    lax.fori_loop(0, n // 128, zb, 0)


_UNROLL = 4


def _scan_chunks(mm, b, base, buf, fn, carry, unroll=_UNROLL):
    lanes = _lanes()
    for ci in range(NCH):
        pltpu.sync_copy(mm.at[b, pl.ds(base + ci * CH, CH)], buf)
        cbase = base + ci * CH

        def vbody(i, c, _cbase=cbase):
            for u in range(unroll):
                off = i * (16 * unroll) + u * 16
                x = buf[pl.ds(off, 16)]
                bits = lax.bitcast_convert_type(x, I32)
                gidx = _cbase + off + lanes
                c = fn(bits, gidx, c)
            return c

        carry = lax.fori_loop(0, VPC // unroll, vbody, carry)
    return carry


def _scan_chunks_dyn(mm, b, base, buf, fn, carry):
    lanes = _lanes()

    def cbody(ci, c):
        pltpu.sync_copy(mm.at[b, pl.ds(base + ci * CH, CH)], buf)
        cbase = base + ci * CH

        def vbody(i, cc):
            x = buf[pl.ds(i * 16, 16)]
            bits = lax.bitcast_convert_type(x, I32)
            gidx = cbase + i * 16 + lanes
            return fn(bits, gidx, cc)

        return lax.fori_loop(0, VPC, vbody, c)

    return lax.fori_loop(0, NCH, cbody, carry)


def _scan_coll(colb, coli, n, fn, carry, unroll=_UNROLL):
    lanes = _lanes()

    def vbody(i, c):
        for u in range(unroll):
            off = i * (16 * unroll) + u * 16
            x = colb[pl.ds(off, 16)]
            gidx = coli[pl.ds(off, 16)]
            valid = (off + lanes) < n
            c = fn(x, gidx, valid, c)
        return c

    nv = (n + 16 * unroll - 1) // (16 * unroll)
    return lax.fori_loop(0, nv, vbody, carry)


_CAP = 24624


def _sc_body(mm, pts0, pts1, out, buf, bufb, hist, colb, coli, eqi, cand2,
             allc, rowbuf, mrg, pts0v, pts1v, smem, sema, semb,
             cands_s, rows_s):
    lanes = _lanes()
    ones = jnp.full((16,), 1, I32)
    zero = jnp.zeros((), I32)
    cc = lax.axis_index("c")
    ss = lax.axis_index("s")
    grp = ss // 4
    mem = ss % 4
    b = cc * 4 + grp
    base = mem * QUART

    pltpu.sync_copy(pts0.at[b], pts0v)
    pltpu.sync_copy(pts1.at[b], pts1v)

    with jax.named_scope("p1_fused"):
        _zero_vmem(hist, 32768)

        def compact_fn(bits, gidx, capv, c):
            n_st, ngev = c
            mge = bits >= tbits
            mst = jnp.logical_and(mge, capv)
            plsc.store_compressed(colb.at[pl.ds(n_st, 16)], bits, mask=mst)
            plsc.store_compressed(coli.at[pl.ds(n_st, 16)], gidx, mask=mst)
            n_st = n_st + plsc.all_reduce_population_count(mst)[0]
            ngev = ngev + mge.astype(I32)
            return n_st, ngev

        bufs = (buf, bufb)
        sems = (sema, semb)

        def _issue(ci):
            return pltpu.async_copy(
                mm.at[b, pl.ds(base + ci * CH, CH)], bufs[ci % 2],
                sems[ci % 2])

        _issue(0).wait()
        pending = _issue(1)

        def ph(i, c):
            for u in range(_UNROLL):
                off = i * (16 * _UNROLL) + u * 16
                x = buf[pl.ds(off, 16)]
                bits = lax.bitcast_convert_type(x, I32)
                d1 = lax.shift_right_logical(bits, 19)
                plsc.addupdate_scatter(hist, [lanes * 2048 + d1], ones)
            return c

        lax.fori_loop(0, VPC // _UNROLL, ph, 0)
        tb, _ = _threshold_find(hist, 2048, 64)
        tbits = tb * 524288

        zvec = jnp.zeros((16,), I32)
        carry = (zero, zvec)
        for ci in range(NCH):
            cur = bufs[ci % 2]
            if ci >= 1:
                pending.wait()
                if ci + 1 < NCH:
                    pending = _issue(ci + 1)
            cbase = base + ci * CH
            capv = jnp.full((16,), carry[0], I32) < _CAP

            def fb(i, c, _cbase=cbase, _capv=capv, _buf=cur):
                for u in range(_UNROLL):
                    off = i * (16 * _UNROLL) + u * 16
                    x = _buf[pl.ds(off, 16)]
                    bits = lax.bitcast_convert_type(x, I32)
                    c = compact_fn(bits, _cbase + off + lanes, _capv, c)
                return c

            carry = lax.fori_loop(0, VPC // _UNROLL, fb, carry)

        n_st, ngev = carry
        n_ge = jnp.sum(ngev)
        ok = jnp.logical_and(n_st == n_ge, n_ge >= 128)

    _zero_vmem(hist, 32768)

    @pl.when(ok)
    def _():
        def f1(x, gidx, valid, c):
            d1 = lax.shift_right_logical(x, 19)
            plsc.addupdate_scatter(hist, [lanes * 2048 + d1], ones,
                                   mask=valid)
            return c

        _scan_coll(colb, coli, n_st, f1, 0)

    @pl.when(jnp.logical_not(ok))
    def _():
        def p1f(bits, gidx, c):
            d1 = lax.shift_right_logical(bits, 19)
            plsc.addupdate_scatter(hist, [lanes * 2048 + d1], ones)
            return c

        _scan_chunks_dyn(mm, b, base, buf, p1f, 0)

    b1, cnt1 = _threshold_find(hist, 2048, 128)
    tgt2 = 128 - cnt1

    _zero_vmem(hist, 32768)

    @pl.when(ok)
    def _():
        def f2(x, gidx, valid, c):
            msk = jnp.logical_and(
                valid, lax.shift_right_logical(x, 19) == b1)
            d2 = jnp.bitwise_and(lax.shift_right_logical(x, 8), 0x7FF)
            plsc.addupdate_scatter(hist, [lanes * 2048 + d2], ones,
                                   mask=msk)
            return c

        _scan_coll(colb, coli, n_st, f2, 0)

    @pl.when(jnp.logical_not(ok))
    def _():
        def p2(bits, gidx, c):
            msk = lax.shift_right_logical(bits, 19) == b1
            d2 = jnp.bitwise_and(lax.shift_right_logical(bits, 8), 0x7FF)
            plsc.addupdate_scatter(hist, [lanes * 2048 + d2], ones,
                                   mask=msk)
            return c

        _scan_chunks_dyn(mm, b, base, buf, p2, 0)

    b2, cnt2 = _threshold_find(hist, 2048, tgt2)
    tgt3 = tgt2 - cnt2
    p20 = b1 * 2048 + b2

    _zero_vmem(hist, 4096)

    @pl.when(ok)
    def _():
        def f3(x, gidx, valid, c):
            msk = jnp.logical_and(
                valid, lax.shift_right_logical(x, 8) == p20)
            d3 = jnp.bitwise_and(x, 0xFF)
            plsc.addupdate_scatter(hist, [lanes * 256 + d3], ones,
                                   mask=msk)
            return c

        _scan_coll(colb, coli, n_st, f3, 0)

    @pl.when(jnp.logical_not(ok))
    def _():
        def p3(bits, gidx, c):
            msk = lax.shift_right_logical(bits, 8) == p20
            d3 = jnp.bitwise_and(bits, 0xFF)
            plsc.addupdate_scatter(hist, [lanes * 256 + d3], ones,
                                   mask=msk)
            return c

        _scan_chunks_dyn(mm, b, base, buf, p3, 0)

    b3, _ = _threshold_find(hist, 256, tgt3)
    thr = p20 * 256 + b3

    neg1 = jnp.full((16,), -1, I32)
    for v in range(10):
        cand2[pl.ds(v * 16, 16)] = neg1
        cand2[pl.ds(160 + v * 16, 16)] = 0x7FF00000 + v * 16 + lanes

    def p4(bits, gidx, valid, c):
        gt_off, eq_off = c
        mg = jnp.logical_and(valid, bits > thr)
        plsc.store_compressed(cand2.at[pl.ds(gt_off, 16)], bits, mask=mg)
        plsc.store_compressed(cand2.at[pl.ds(160 + gt_off, 16)], gidx,
                              mask=mg)
        gt_off = gt_off + jnp.sum(mg.astype(I32))
        cap = jnp.full((16,), eq_off, I32) < 128
        me = jnp.logical_and(jnp.logical_and(valid, bits == thr), cap)
        plsc.store_compressed(eqi.at[pl.ds(eq_off, 16)], gidx, mask=me)
        eq_off = eq_off + jnp.sum(me.astype(I32))
        return gt_off, eq_off

    @pl.when(ok)
    def _():
        gt_off, _eq = _scan_coll(colb, coli, n_st, p4, (zero, zero))
        smem[0] = gt_off

    @pl.when(jnp.logical_not(ok))
    def _():
        def p4f(bits, gidx, c):
            tv = jnp.full((16,), 1, I32) > 0
            return p4(bits, gidx, tv, c)

        gt_off, _eq = _scan_chunks_dyn(mm, b, base, buf, p4f, (zero, zero))
        smem[0] = gt_off

    count_gt = smem[0]
    need_eq = 128 - count_gt

    thr_vec = jnp.full((16,), thr, I32)
    for v in range(8):
        @pl.when(v * 16 < need_eq)
        def _(v=v):
            kk = need_eq - v * 16
            msk = lanes < kk
            ev = eqi[pl.ds(v * 16, 16)]
            plsc.store_compressed(cand2.at[pl.ds(count_gt + v * 16, 16)],
                                  thr_vec, mask=msk)
            plsc.store_compressed(
                cand2.at[pl.ds(160 + count_gt + v * 16, 16)], ev, mask=msk)

    zf = jnp.zeros((16,), jnp.float32)

    def zr(i, c):
        rowbuf[pl.ds(i * 16, 16)] = zf
        return c

    lax.fori_loop(0, 80, zr, 0)

    pltpu.sync_copy(cand2, cands_s.at[pl.ds(grp * 1280 + mem * 320, 320)])
    plsc.subcore_barrier()

    pltpu.sync_copy(cands_s.at[pl.ds(grp * 1280, 1280)], allc)

    def rank_body(v, c):
        ob = allc[pl.ds(mem * 320 + v * 16, 16)]
        oi = allc[pl.ds(mem * 320 + 160 + v * 16, 16)]

        def jt_body(jt, r):
            def w_body(w, r2):
                cb = allc[pl.ds(jt * 320 + w * 16, 16)]
                cv = allc[pl.ds(jt * 320 + 160 + w * 16, 16)]
                for k in range(16):
                    perm = jnp.bitwise_and(lanes + k, 15)
                    rb = cb.at[perm].get(mode="promise_in_bounds",
                                         unique_indices=True)
                    ri = cv.at[perm].get(mode="promise_in_bounds",
                                         unique_indices=True)
                    better = jnp.logical_or(
                        rb > ob,
                        jnp.logical_and(rb == ob, ri < oi))
                    r2 = r2 + better.astype(I32)
                return r2

            return lax.fori_loop(0, 8, w_body, r)

        r = lax.fori_loop(0, 4, jt_body, jnp.zeros((16,), I32))

        safe_i = jnp.where(ob >= 0, oi, 0)
        ii = lax.shift_right_logical(safe_i, 10)
        jj = jnp.bitwise_and(safe_i, 1023)
        tgt = jnp.where(r < 128, r,
                        128 + jnp.bitwise_and(v * 16 + lanes, 31))
        t8 = tgt * 8
        for d in range(3):
            dcol = jnp.full((16,), d, I32)
            g0 = plsc.load_gather(pts0v, [ii * 3 + d])
            plsc.store_scatter(rowbuf, [t8 + d], g0)
            g1 = plsc.load_gather(pts1v, [jj * 3 + d])
            plsc.store_scatter(rowbuf, [t8 + 4 + d], g1)
        return c

    lax.fori_loop(0, 8, rank_body, 0)

    slot = (grp * 4 + mem) * 1280
    pltpu.sync_copy(rowbuf, rows_s.at[pl.ds(slot, 1280)])
    plsc.subcore_barrier()

    @pl.when(mem == 0)
    def _():
        for t in range(1, 4):
            pltpu.sync_copy(rows_s.at[pl.ds((grp * 4 + t) * 1280, 1280)],
                            mrg)

            def madd(i, c):
                rowbuf[pl.ds(i * 16, 16)] = (rowbuf[pl.ds(i * 16, 16)]
                                             + mrg[pl.ds(i * 16, 16)])
                return c

            lax.fori_loop(0, 64, madd, 0)
        pltpu.sync_copy(rowbuf.at[pl.ds(0, 1024)], out.at[b])


def _sc_topk_anchors(mm, pts0f, pts1f):
    mesh = plsc.VectorSubcoreMesh(core_axis_name="c", subcore_axis_name="s")
    fn = pl.kernel(
        _sc_body,
        out_type=jax.ShapeDtypeStruct((N, A * 8), jnp.float32),
        mesh=mesh,
        scratch_types=[
            pltpu.VMEM((CH,), jnp.float32),
            pltpu.VMEM((CH,), jnp.float32),
            pltpu.VMEM((32768,), I32),
            pltpu.VMEM((32832,), I32),
            pltpu.VMEM((32832,), I32),
            pltpu.VMEM((160,), I32),
            pltpu.VMEM((320,), I32),
            pltpu.VMEM((1280,), I32),
            pltpu.VMEM((1280,), jnp.float32),
            pltpu.VMEM((1280,), jnp.float32),
            pltpu.VMEM((3 * L,), jnp.float32),
            pltpu.VMEM((3 * L,), jnp.float32),
            pltpu.SMEM((8,), I32),
            pltpu.SemaphoreType.DMA,
            pltpu.SemaphoreType.DMA,
            pltpu.VMEM_SHARED((5120,), I32),
            pltpu.VMEM_SHARED((16 * 1280,), jnp.float32),
        ],
        compiler_params=pltpu.CompilerParams(needs_layout_passes=False),
    )
    return fn(mm, pts0f, pts1f)


def _dense_body(p_ref, a_ref, o0_ref, o1_ref):
    p = p_ref[0]
    a = a_ref[0]
    for side, o_ref in ((0, o0_ref), (1, o1_ref)):
        diffs = []
        for c in range(3):
            prow = p[4 * side + c: 4 * side + c + 1, :]
            acol = a[:, 4 * side + c: 4 * side + c + 1]
            diffs.append(prow - acol)
        dist = jnp.sqrt(diffs[0] * diffs[0] + diffs[1] * diffs[1]
                        + diffs[2] * diffs[2])
        feats = diffs + [dist]
        for d, f in enumerate(feats):
            norm = jnp.sum(jnp.abs(f), axis=0, keepdims=True)
            o_ref[0, d * A:(d + 1) * A, :] = f / norm


def _dense_call(P, anchors):
    return pl.pallas_call(
        _dense_body,
        grid=(N,),
        in_specs=[
            pl.BlockSpec((1, 8, L), lambda b: (b, 0, 0)),
            pl.BlockSpec((1, A, 8), lambda b: (b, 0, 0)),
        ],
        out_specs=[
            pl.BlockSpec((1, 4 * A, L), lambda b: (b, 0, 0)),
            pl.BlockSpec((1, 4 * A, L), lambda b: (b, 0, 0)),
        ],
        out_shape=[
            jax.ShapeDtypeStruct((N, 4 * A, L), jnp.float32),
            jax.ShapeDtypeStruct((N, 4 * A, L), jnp.float32),
        ],
    )(P, anchors)


def kernel(match_mask, pts_3d0, pts_3d1, K0, K1, non_epipolar):
    mm = match_mask.reshape(N, FLAT)
    anchors = _sc_topk_anchors(mm, pts_3d0.reshape(N, 3 * L),
                               pts_3d1.reshape(N, 3 * L))
    anchors = anchors.reshape(N, A, 8)
    z2 = jnp.zeros((N, 1, L), jnp.float32)
    P = jnp.concatenate(
        [pts_3d0.transpose(0, 2, 1), z2, pts_3d1.transpose(0, 2, 1), z2],
        axis=1)
    out0, out1 = _dense_call(P, anchors)
    return (out0.reshape(N, 4 * A, H, W), out1.reshape(N, 4 * A, H, W))
